# Initial kernel scaffold; baseline (speedup 1.0000x reference)
import jax
import jax.numpy as jnp
from jax.experimental import pallas as pl


def kernel(x, edge_index, edge_attr, batch, Wl1, Wr1, We1, att1, b1, Wl2, Wr2, We2, att2, b2, W3, b3, gamma, beta, W4, b4):
    raise NotImplementedError("write your pallas kernel here")


# speedup vs baseline: 13.9515x; 13.9515x over previous
import functools

import jax
import jax.numpy as jnp
from jax import lax
from jax.experimental import pallas as pl
from jax.experimental.pallas import tpu as pltpu
from jax.experimental.pallas import tpu_sc as plsc

N = 50000
E = 800000
G = 512
H = 4

NC = 2
NS = 16
NW = NC * NS

W = 128
NP = 53248
RPT = NP // NS
RPW = NP // NW

E0PT = 25088
E0P = E0PT * NW
EPT = 26624
EP = EPT * NW

GP = 544

_MESH = plsc.VectorSubcoreMesh(core_axis_name="c", subcore_axis_name="s",
                               num_cores=NC, num_subcores=NS)
_SC_PARAMS = pltpu.CompilerParams(use_tc_tiling_on_sc=False,
                                  needs_layout_passes=False)


def _ids():
    c = lax.axis_index("c")
    s = lax.axis_index("s")
    return c, s, s * NC + c


def _zero_rows(zb, n_lanes):
    z = jnp.zeros((16,), jnp.float32)
    def body(r, _):
        for jj in range(n_lanes // 16):
            zb[r, pl.ds(jj * 16, 16)] = z
        return 0
    lax.fori_loop(0, zb.shape[0], body, 0)


def _zero_spmem(zb, spm, s):
    _zero_rows(zb, zb.shape[1])
    for k in range(RPT // 832):
        pltpu.sync_copy(zb, spm.at[pl.ds(s * RPT + k * 832, 832)])


def _p0_body(dst_h, eaf_h, sums_h, dst_v, ea_v, row_v, zb_v, spm, sem):
    c, s, wid = _ids()
    _zero_spmem(zb_v, spm, s)
    plsc.subcore_barrier()

    iota = lax.iota(jnp.int32, 16)

    def win(w, _):
        base = wid * E0PT + w * W
        pltpu.sync_copy(dst_h.at[pl.ds(base, W)], dst_v)
        pltpu.sync_copy(eaf_h.at[pl.ds(base * 4, W * 4)], ea_v.at[pl.ds(0, W * 4)])

        def edge(e, _):
            av = ea_v[pl.ds(e * 4, 16)]
            row = jnp.where(iota < 4, av, 0.0)
            row = jnp.where(iota == 4, 1.0, row)
            row_v[e, pl.ds(0, 16)] = row
            return 0

        lax.fori_loop(0, W, edge, 0)
        pltpu.sync_copy(row_v, spm.at[dst_v], add=True)
        return 0

    lax.fori_loop(0, E0PT // W, win, 0)
    plsc.subcore_barrier()
    pltpu.sync_copy(spm.at[pl.ds(s * RPT, RPT)],
                    sums_h.at[c, pl.ds(s * RPT, RPT)])


@functools.partial(
    pl.kernel,
    out_type=jax.ShapeDtypeStruct((NC, NP, 16), jnp.float32),
    mesh=_MESH,
    compiler_params=_SC_PARAMS,
    scratch_types=[
        pltpu.VMEM((W,), jnp.int32),
        pltpu.VMEM((W * 4 + 16,), jnp.float32),
        pltpu.VMEM((W, 16), jnp.float32),
        pltpu.VMEM((832, 16), jnp.float32),
        pltpu.MemorySpace.VMEM_SHARED((NP, 16), jnp.float32),
        pltpu.SemaphoreType.DMA,
    ],
)
def _p0(dst_h, eaf_h, sums_h, *rest):
    _p0_body(dst_h, eaf_h, sums_h, *rest)


def _p1_body(HC, C, xl_h, xr_h, src_h, dst_h, eaf_h, we_h, att_h,
             ex_h, den_h, src_v, dst_v, ea_v, xl_v, xr_v, ex_v,
             exb_v, we_v, att_v, zb_v, spm, sem):
    c, s, wid = _ids()
    C16 = C // 16
    _zero_spmem(zb_v, spm, s)
    pltpu.sync_copy(we_h, we_v)
    pltpu.sync_copy(att_h, att_v)
    plsc.subcore_barrier()

    iota = lax.iota(jnp.int32, 16)

    def win(w, _):
        base = wid * EPT + w * W
        pltpu.sync_copy(src_h.at[pl.ds(base, W)], src_v)
        pltpu.sync_copy(dst_h.at[pl.ds(base, W)], dst_v)
        pltpu.sync_copy(eaf_h.at[pl.ds(base * 4, W * 4)], ea_v.at[pl.ds(0, W * 4)])
        pltpu.async_copy(xl_h.at[src_v], xl_v, sem).wait()
        pltpu.async_copy(xr_h.at[dst_v], xr_v, sem).wait()

        def edge(e, _):
            av = ea_v[pl.ds(e * 4, 16)]
            a0, a1, a2, a3 = av[0], av[1], av[2], av[3]
            sv = jnp.zeros((16,), jnp.float32)
            for h in range(H):
                sh = jnp.float32(0.0)
                for jj in range(C16):
                    j = h * C16 + jj
                    ef = (a0 * we_v[0, pl.ds(j * 16, 16)]
                          + a1 * we_v[1, pl.ds(j * 16, 16)]
                          + a2 * we_v[2, pl.ds(j * 16, 16)]
                          + a3 * we_v[3, pl.ds(j * 16, 16)])
                    m = xl_v[e, pl.ds(j * 16, 16)] + xr_v[e, pl.ds(j * 16, 16)] + ef
                    t = jnp.maximum(m, 0.2 * m)
                    sh = sh + jnp.sum(att_v[h, pl.ds(jj * 16, 16)] * t)
                sv = jnp.where(iota == h, sh, sv)
            ev = jnp.exp(sv)
            exb_v[e, pl.ds(0, 16)] = jnp.where(iota < 4, ev, 0.0)
            ex_v[pl.ds(e * 4, 16)] = ev
            return 0

        lax.fori_loop(0, W, edge, 0)
        pltpu.sync_copy(ex_v.at[pl.ds(0, W * 4)], ex_h.at[pl.ds(base * 4, W * 4)])
        pltpu.sync_copy(exb_v, spm.at[dst_v], add=True)
        return 0

    lax.fori_loop(0, EPT // W, win, 0)
    plsc.subcore_barrier()
    pltpu.sync_copy(spm.at[pl.ds(s * RPT, RPT)],
                    den_h.at[c, pl.ds(s * RPT, RPT)])


def _make_p1(HC, C):
    @functools.partial(
        pl.kernel,
        out_type=(jax.ShapeDtypeStruct((EP * 4,), jnp.float32),
                  jax.ShapeDtypeStruct((NC, NP, 16), jnp.float32)),
        mesh=_MESH,
        compiler_params=_SC_PARAMS,
        scratch_types=[
            pltpu.VMEM((W,), jnp.int32),
            pltpu.VMEM((W,), jnp.int32),
            pltpu.VMEM((W * 4 + 16,), jnp.float32),
            pltpu.VMEM((W, HC), jnp.float32),
            pltpu.VMEM((W, HC), jnp.float32),
            pltpu.VMEM((W * 4 + 16,), jnp.float32),
            pltpu.VMEM((W, 16), jnp.float32),
            pltpu.VMEM((4, HC), jnp.float32),
            pltpu.VMEM((4, C), jnp.float32),
            pltpu.VMEM((832, 16), jnp.float32),
            pltpu.MemorySpace.VMEM_SHARED((NP, 16), jnp.float32),
            pltpu.SemaphoreType.DMA,
        ],
    )
    def _p1(*args):
        _p1_body(HC, C, *args)
    return _p1


def _p3_body(NSL, CPH, args):
    tabs = args[:NSL]
    src_h, dst_h, exf_h = args[NSL:NSL + 3]
    outs = args[NSL + 3:2 * NSL + 3]
    src_v, dst_v, ex_v, xls_v, sc_v, zb_v, spm, sem = args[2 * NSL + 3:]
    c, s, wid = _ids()
    _zero_rows(zb_v, 16)

    for sl in range(NSL):
        head = sl // CPH
        for k in range(RPT // 832):
            pltpu.sync_copy(zb_v, spm.at[pl.ds(s * RPT + k * 832, 832)])
        plsc.subcore_barrier()

        def win(w, _):
            base = wid * EPT + w * W
            pltpu.sync_copy(src_h.at[pl.ds(base, W)], src_v)
            pltpu.sync_copy(dst_h.at[pl.ds(base, W)], dst_v)
            pltpu.sync_copy(exf_h.at[pl.ds(base * 4, W * 4)],
                            ex_v.at[pl.ds(0, W * 4)])
            pltpu.async_copy(tabs[sl].at[src_v], xls_v, sem).wait()

            def edge(e, _):
                exv = ex_v[pl.ds(e * 4, 16)]
                sc_v[e, pl.ds(0, 16)] = xls_v[e, pl.ds(0, 16)] * exv[head]
                return 0

            lax.fori_loop(0, W, edge, 0)
            pltpu.sync_copy(sc_v, spm.at[dst_v], add=True)
            return 0

        lax.fori_loop(0, EPT // W, win, 0)
        plsc.subcore_barrier()
        pltpu.sync_copy(spm.at[pl.ds(s * RPT, RPT)],
                        outs[sl].at[c, pl.ds(s * RPT, RPT)])


def _make_p3(NSL, CPH):
    @functools.partial(
        pl.kernel,
        out_type=tuple(jax.ShapeDtypeStruct((NC, NP, 16), jnp.float32)
                       for _ in range(NSL)),
        mesh=_MESH,
        compiler_params=_SC_PARAMS,
        scratch_types=[
            pltpu.VMEM((W,), jnp.int32),
            pltpu.VMEM((W,), jnp.int32),
            pltpu.VMEM((W * 4 + 16,), jnp.float32),
            pltpu.VMEM((W, 16), jnp.float32),
            pltpu.VMEM((W, 16), jnp.float32),
            pltpu.VMEM((832, 16), jnp.float32),
            pltpu.MemorySpace.VMEM_SHARED((NP, 16), jnp.float32),
            pltpu.SemaphoreType.DMA,
        ],
    )
    def _p3(*args):
        _p3_body(NSL, CPH, args)
    return _p3


def _pool_body(args):
    parts = args[:8]
    d_h, b_h, b2_h, g_h = args[8:12]
    b_v, b2_v, bufs, d0_v, d1_v, hrow_v, zb_v, spm, sem = args[12:]
    c, s, wid = _ids()
    pltpu.sync_copy(b2_h, b2_v)
    _zero_rows(zb_v, 32)
    pltpu.sync_copy(zb_v.at[pl.ds(0, GP // NS)], spm.at[pl.ds(s * (GP // NS), GP // NS)])
    plsc.subcore_barrier()

    def win(w, _):
        base = wid * RPW + w * W
        pltpu.sync_copy(b_h.at[pl.ds(base, W)], b_v)
        pltpu.sync_copy(d_h.at[0, pl.ds(base, W)], d0_v)
        pltpu.sync_copy(d_h.at[1, pl.ds(base, W)], d1_v)
        for t in range(8):
            for cc in range(NC):
                pltpu.sync_copy(parts[t].at[cc, pl.ds(base, W)], bufs[t * 2 + cc])

        def row(r, _):
            dv = d0_v[r, pl.ds(0, 16)] + d1_v[r, pl.ds(0, 16)]
            rec = 1.0 / (dv + 1e-16)
            for jj in range(2):
                acc = (bufs[2 * jj][r, pl.ds(0, 16)]
                       + bufs[2 * jj + 1][r, pl.ds(0, 16)]) * rec[0]
                for h in range(1, 4):
                    p = h * 2 + jj
                    acc = acc + (bufs[2 * p][r, pl.ds(0, 16)]
                                 + bufs[2 * p + 1][r, pl.ds(0, 16)]) * rec[h]
                hrow_v[r, pl.ds(jj * 16, 16)] = acc * 0.25 + b2_v[pl.ds(jj * 16, 16)]
            return 0

        lax.fori_loop(0, W, row, 0)
        pltpu.sync_copy(hrow_v, spm.at[b_v], add=True)
        return 0

    lax.fori_loop(0, RPW // W, win, 0)
    plsc.subcore_barrier()
    pltpu.sync_copy(spm.at[pl.ds(s * (GP // NS), GP // NS)],
                    g_h.at[c, pl.ds(s * (GP // NS), GP // NS)])


@functools.partial(
    pl.kernel,
    out_type=jax.ShapeDtypeStruct((NC, GP, 32), jnp.float32),
    mesh=_MESH,
    compiler_params=_SC_PARAMS,
    scratch_types=[
        pltpu.VMEM((W,), jnp.int32),
        pltpu.VMEM((32,), jnp.float32),
        [pltpu.VMEM((W, 16), jnp.float32)] * 16,
        pltpu.VMEM((W, 16), jnp.float32),
        pltpu.VMEM((W, 16), jnp.float32),
        pltpu.VMEM((W, 32), jnp.float32),
        pltpu.VMEM((832, 32), jnp.float32),
        pltpu.MemorySpace.VMEM_SHARED((GP, 32), jnp.float32),
        pltpu.SemaphoreType.DMA,
    ],
)
def _pool(*args):
    _pool_body(args)


_R = 512
_GRID = NP // _R


def _tc1_body(x_ref, wl_ref, wr_ref, s0_ref, s1_ref,
              xl_ref, t0_ref, t1_ref, t2_ref, t3_ref, xr_ref, la_ref):
    xb = x_ref[...]
    xl = jnp.dot(xb, wl_ref[...], preferred_element_type=jnp.float32)
    xl_ref[...] = xl
    for h, t_ref in enumerate((t0_ref, t1_ref, t2_ref, t3_ref)):
        t_ref[...] = xl[:, 16 * h:16 * h + 16]
    xr_ref[...] = jnp.dot(xb, wr_ref[...], preferred_element_type=jnp.float32)
    ssum = s0_ref[...] + s1_ref[...]
    la_ref[...] = ssum / jnp.clip(ssum[:, 4:5], 1.0, None)


def _tc1(x_p, Wl1, Wr1, sums):
    return pl.pallas_call(
        _tc1_body,
        grid=(_GRID,),
        in_specs=[
            pl.BlockSpec((_R, 32), lambda i: (i, 0)),
            pl.BlockSpec((32, 64), lambda i: (0, 0)),
            pl.BlockSpec((32, 64), lambda i: (0, 0)),
            pl.BlockSpec((_R, 16), lambda i: (i, 0)),
            pl.BlockSpec((_R, 16), lambda i: (i, 0)),
        ],
        out_specs=[pl.BlockSpec((_R, 64), lambda i: (i, 0))]
        + [pl.BlockSpec((_R, 16), lambda i: (i, 0))] * 4
        + [pl.BlockSpec((_R, 64), lambda i: (i, 0)),
           pl.BlockSpec((_R, 16), lambda i: (i, 0))],
        out_shape=[jax.ShapeDtypeStruct((NP, 64), jnp.float32)]
        + [jax.ShapeDtypeStruct((NP, 16), jnp.float32)] * 4
        + [jax.ShapeDtypeStruct((NP, 64), jnp.float32),
           jax.ShapeDtypeStruct((NP, 16), jnp.float32)],
    )(x_p, Wl1, Wr1, sums[0], sums[1])


def _tc2_body(p0_ref, p1_ref, p2_ref, p3_ref, d_ref, b1_ref, wl_ref, wr_ref,
              xl_ref, *out_refs):
    d = d_ref[0] + d_ref[1]
    segs = [(p_ref[0] + p_ref[1]) / (d[:, h:h + 1] + 1e-16)
            for h, p_ref in enumerate((p0_ref, p1_ref, p2_ref, p3_ref))]
    h = jnp.maximum(jnp.concatenate(segs, axis=1) + b1_ref[...], 0.0)
    xl = jnp.dot(h, wl_ref[...], preferred_element_type=jnp.float32)
    xl_ref[...] = xl
    for t in range(8):
        out_refs[t][...] = xl[:, 16 * t:16 * t + 16]
    out_refs[8][...] = jnp.dot(h, wr_ref[...], preferred_element_type=jnp.float32)


def _tc2(o1, den1, b1, Wl2, Wr2):
    return pl.pallas_call(
        _tc2_body,
        grid=(_GRID,),
        in_specs=[pl.BlockSpec((2, _R, 16), lambda i: (0, i, 0))] * 5
        + [
            pl.BlockSpec((1, 64), lambda i: (0, 0)),
            pl.BlockSpec((64, 128), lambda i: (0, 0)),
            pl.BlockSpec((64, 128), lambda i: (0, 0)),
        ],
        out_specs=[pl.BlockSpec((_R, 128), lambda i: (i, 0))]
        + [pl.BlockSpec((_R, 16), lambda i: (i, 0))] * 8
        + [pl.BlockSpec((_R, 128), lambda i: (i, 0))],
        out_shape=[jax.ShapeDtypeStruct((NP, 128), jnp.float32)]
        + [jax.ShapeDtypeStruct((NP, 16), jnp.float32)] * 8
        + [jax.ShapeDtypeStruct((NP, 128), jnp.float32)],
    )(o1[0], o1[1], o1[2], o1[3], den1, b1.reshape(1, 64), Wl2, Wr2)


def _tc4_body(g_ref, w3_ref, b3_ref, gm_ref, bt_ref, w4_ref, b4_ref, o_ref):
    g = g_ref[0, :G, :] + g_ref[1, :G, :]
    h = jnp.maximum(jnp.dot(g, w3_ref[...], preferred_element_type=jnp.float32)
                    + b3_ref[...], 0.0)
    mu = jnp.mean(h, axis=-1, keepdims=True)
    var = jnp.mean((h - mu) ** 2, axis=-1, keepdims=True)
    hn = (h - mu) * jax.lax.rsqrt(var + 1e-5) * gm_ref[...] + bt_ref[...]
    o_ref[...] = jnp.dot(hn, w4_ref[...], preferred_element_type=jnp.float32) + b4_ref[...]


def _tc4(gacc, W3, b3, gamma, beta, W4, b4):
    return pl.pallas_call(
        _tc4_body,
        out_shape=jax.ShapeDtypeStruct((G, 64), jnp.float32),
    )(gacc, W3, b3.reshape(1, 128), gamma.reshape(1, 128),
      beta.reshape(1, 128), W4, b4.reshape(1, 64))


_P1_L1 = _make_p1(64, 16)
_P1_L2 = _make_p1(128, 32)
_P3_L1 = _make_p3(4, 1)
_P3_L2 = _make_p3(8, 2)


def kernel(x, edge_index, edge_attr, batch, Wl1, Wr1, We1, att1, b1,
           Wl2, Wr2, We2, att2, b2, W3, b3, gamma, beta, W4, b4):
    src0 = edge_index[0]
    dst0 = edge_index[1]

    pad0 = E0P - E
    dst0_p = jnp.concatenate([dst0, N + (jnp.arange(pad0, dtype=jnp.int32) % 16)])
    ea0_f = jnp.concatenate(
        [edge_attr, jnp.zeros((pad0, 4), jnp.float32)]).reshape(-1)

    x_p = jnp.pad(x, ((0, NP - N), (0, 0)))

    sums = _p0(dst0_p, ea0_f)
    xl1, t0, t1, t2, t3, xr1, la = _tc1(x_p, Wl1, Wr1, sums)

    loop_attr = la[:N, :4]
    pad1 = EP - E - N
    loop_idx = jnp.arange(N, dtype=jnp.int32)
    pad_idx = N + (jnp.arange(pad1, dtype=jnp.int32) % 16)
    src_p = jnp.concatenate([src0, loop_idx, pad_idx])
    dst_p = jnp.concatenate([dst0, loop_idx, pad_idx])
    ea_f = jnp.concatenate(
        [edge_attr, loop_attr, jnp.zeros((pad1, 4), jnp.float32)]).reshape(-1)

    ex1, den1 = _P1_L1(xl1, xr1, src_p, dst_p, ea_f, We1, att1)
    o1 = _P3_L1(t0, t1, t2, t3, src_p, dst_p, ex1)

    tc2_outs = _tc2(o1, den1, b1, Wl2, Wr2)
    xl2, xr2 = tc2_outs[0], tc2_outs[9]
    tabs2 = tc2_outs[1:9]

    ex2, den2 = _P1_L2(xl2, xr2, src_p, dst_p, ea_f, We2, att2)
    o2 = _P3_L2(*tabs2, src_p, dst_p, ex2)

    batch_p = jnp.concatenate(
        [batch, G + (jnp.arange(NP - N, dtype=jnp.int32) % 32)])
    gacc = _pool(*o2, den2, batch_p, b2)
    return _tc4(gacc, W3, b3, gamma, beta, W4, b4)


# speedup vs baseline: 17.4391x; 1.2500x over previous
import functools

import jax
import jax.numpy as jnp
from jax import lax
from jax.experimental import pallas as pl
from jax.experimental.pallas import tpu as pltpu
from jax.experimental.pallas import tpu_sc as plsc

N = 50000
E = 800000
G = 512
H = 4

NC = 2
NS = 16
NW = NC * NS

W = 128
NP = 53248
RPT = NP // NS
RPW = NP // NW

E0PT = 25088
E0P = E0PT * NW
EPT = 26624
EP = EPT * NW

GP = 544

_MESH = plsc.VectorSubcoreMesh(core_axis_name="c", subcore_axis_name="s",
                               num_cores=NC, num_subcores=NS)
_SC_PARAMS = pltpu.CompilerParams(use_tc_tiling_on_sc=False,
                                  needs_layout_passes=False)


def _ids():
    c = lax.axis_index("c")
    s = lax.axis_index("s")
    return c, s, s * NC + c


def _zero_rows(zb, n_lanes):
    z = jnp.zeros((16,), jnp.float32)
    def body(r, _):
        for jj in range(n_lanes // 16):
            zb[r, pl.ds(jj * 16, 16)] = z
        return 0
    lax.fori_loop(0, zb.shape[0], body, 0)


def _zero_spmem(zb, spm, s):
    _zero_rows(zb, zb.shape[1])
    for k in range(RPT // 832):
        pltpu.sync_copy(zb, spm.at[pl.ds(s * RPT + k * 832, 832)])


def _p0_body(dst_h, eaf_h, sums_h, dst_v, ea_v, row_v, zb_v, spm, sem):
    c, s, wid = _ids()
    _zero_spmem(zb_v, spm, s)
    plsc.subcore_barrier()

    iota = lax.iota(jnp.int32, 16)

    def win(w, _):
        base = wid * E0PT + w * W
        pltpu.sync_copy(dst_h.at[pl.ds(base, W)], dst_v)
        pltpu.sync_copy(eaf_h.at[pl.ds(base * 4, W * 4)], ea_v.at[pl.ds(0, W * 4)])

        def edge(e, _):
            av = ea_v[pl.ds(e * 4, 16)]
            row = jnp.where(iota < 4, av, 0.0)
            row = jnp.where(iota == 4, 1.0, row)
            row_v[e, pl.ds(0, 16)] = row
            return 0

        lax.fori_loop(0, W, edge, 0)
        pltpu.sync_copy(row_v, spm.at[dst_v], add=True)
        return 0

    lax.fori_loop(0, E0PT // W, win, 0)
    plsc.subcore_barrier()
    pltpu.sync_copy(spm.at[pl.ds(s * RPT, RPT)],
                    sums_h.at[c, pl.ds(s * RPT, RPT)])


@functools.partial(
    pl.kernel,
    out_type=jax.ShapeDtypeStruct((NC, NP, 16), jnp.float32),
    mesh=_MESH,
    compiler_params=_SC_PARAMS,
    scratch_types=[
        pltpu.VMEM((W,), jnp.int32),
        pltpu.VMEM((W * 4 + 16,), jnp.float32),
        pltpu.VMEM((W, 16), jnp.float32),
        pltpu.VMEM((832, 16), jnp.float32),
        pltpu.MemorySpace.VMEM_SHARED((NP, 16), jnp.float32),
        pltpu.SemaphoreType.DMA,
    ],
)
def _p0(dst_h, eaf_h, sums_h, *rest):
    _p0_body(dst_h, eaf_h, sums_h, *rest)


def _p1_body(HC, C, xl_h, xr_h, src_h, dst_h, eaf_h, we_h, att_h,
             ex_h, den_h, src_v, dst_v, ea_v, xl_v, xr_v, ex_v,
             exb_v, we_v, att_v, zb_v, spm, sem):
    c, s, wid = _ids()
    C16 = C // 16
    _zero_spmem(zb_v, spm, s)
    pltpu.sync_copy(we_h, we_v)
    pltpu.sync_copy(att_h, att_v)
    plsc.subcore_barrier()

    iota = lax.iota(jnp.int32, 16)

    def win(w, _):
        base = wid * EPT + w * W
        pltpu.sync_copy(src_h.at[pl.ds(base, W)], src_v)
        pltpu.sync_copy(dst_h.at[pl.ds(base, W)], dst_v)
        pltpu.sync_copy(eaf_h.at[pl.ds(base * 4, W * 4)], ea_v.at[pl.ds(0, W * 4)])
        pltpu.async_copy(xl_h.at[src_v], xl_v, sem).wait()
        pltpu.async_copy(xr_h.at[dst_v], xr_v, sem).wait()

        def edge(e, _):
            av = ea_v[pl.ds(e * 4, 16)]
            a0, a1, a2, a3 = av[0], av[1], av[2], av[3]
            sv = jnp.zeros((16,), jnp.float32)
            for h in range(H):
                sh = jnp.float32(0.0)
                for jj in range(C16):
                    j = h * C16 + jj
                    ef = (a0 * we_v[0, pl.ds(j * 16, 16)]
                          + a1 * we_v[1, pl.ds(j * 16, 16)]
                          + a2 * we_v[2, pl.ds(j * 16, 16)]
                          + a3 * we_v[3, pl.ds(j * 16, 16)])
                    m = xl_v[e, pl.ds(j * 16, 16)] + xr_v[e, pl.ds(j * 16, 16)] + ef
                    t = jnp.maximum(m, 0.2 * m)
                    sh = sh + jnp.sum(att_v[h, pl.ds(jj * 16, 16)] * t)
                sv = jnp.where(iota == h, sh, sv)
            ev = jnp.exp(sv)
            exb_v[e, pl.ds(0, 16)] = jnp.where(iota < 4, ev, 0.0)
            ex_v[pl.ds(e * 4, 16)] = ev
            return 0

        lax.fori_loop(0, W, edge, 0)
        pltpu.sync_copy(ex_v.at[pl.ds(0, W * 4)], ex_h.at[pl.ds(base * 4, W * 4)])
        pltpu.sync_copy(exb_v, spm.at[dst_v], add=True)
        return 0

    lax.fori_loop(0, EPT // W, win, 0)
    plsc.subcore_barrier()
    pltpu.sync_copy(spm.at[pl.ds(s * RPT, RPT)],
                    den_h.at[c, pl.ds(s * RPT, RPT)])


def _make_p1(HC, C):
    @functools.partial(
        pl.kernel,
        out_type=(jax.ShapeDtypeStruct((EP * 4,), jnp.float32),
                  jax.ShapeDtypeStruct((NC, NP, 16), jnp.float32)),
        mesh=_MESH,
        compiler_params=_SC_PARAMS,
        scratch_types=[
            pltpu.VMEM((W,), jnp.int32),
            pltpu.VMEM((W,), jnp.int32),
            pltpu.VMEM((W * 4 + 16,), jnp.float32),
            pltpu.VMEM((W, HC), jnp.float32),
            pltpu.VMEM((W, HC), jnp.float32),
            pltpu.VMEM((W * 4 + 16,), jnp.float32),
            pltpu.VMEM((W, 16), jnp.float32),
            pltpu.VMEM((4, HC), jnp.float32),
            pltpu.VMEM((4, C), jnp.float32),
            pltpu.VMEM((832, 16), jnp.float32),
            pltpu.MemorySpace.VMEM_SHARED((NP, 16), jnp.float32),
            pltpu.SemaphoreType.DMA,
        ],
    )
    def _p1(*args):
        _p1_body(HC, C, *args)
    return _p1


def _p3_body(NSL, CPH, args):
    tabs = args[:NSL]
    src_h, dst_h, exf_h = args[NSL:NSL + 3]
    outs = args[NSL + 3:2 * NSL + 3]
    src_v, dst_v, ex_v, xls_v, sc_v, zb_v, spm, sem = args[2 * NSL + 3:]
    c, s, wid = _ids()
    _zero_rows(zb_v, 16)

    for sl in range(NSL):
        head = sl // CPH
        for k in range(RPT // 832):
            pltpu.sync_copy(zb_v, spm.at[pl.ds(s * RPT + k * 832, 832)])
        plsc.subcore_barrier()

        def win(w, _):
            base = wid * EPT + w * W
            pltpu.sync_copy(src_h.at[pl.ds(base, W)], src_v)
            pltpu.sync_copy(dst_h.at[pl.ds(base, W)], dst_v)
            pltpu.sync_copy(exf_h.at[pl.ds(base * 4, W * 4)],
                            ex_v.at[pl.ds(0, W * 4)])
            pltpu.async_copy(tabs[sl].at[src_v], xls_v, sem).wait()

            def edge(e, _):
                exv = ex_v[pl.ds(e * 4, 16)]
                sc_v[e, pl.ds(0, 16)] = xls_v[e, pl.ds(0, 16)] * exv[head]
                return 0

            lax.fori_loop(0, W, edge, 0)
            pltpu.sync_copy(sc_v, spm.at[dst_v], add=True)
            return 0

        lax.fori_loop(0, EPT // W, win, 0)
        plsc.subcore_barrier()
        pltpu.sync_copy(spm.at[pl.ds(s * RPT, RPT)],
                        outs[sl].at[c, pl.ds(s * RPT, RPT)])


def _p3c_body(args):
    xl_h, dm_h, src_h, dst_h, exf_h = args[:5]
    outs = args[5:7]
    (src_v, dst_v, ex_v, den_v, xl_v, sc_v, zb_v, spm, sem) = args[7:]
    c, s, wid = _ids()
    _zero_rows(zb_v, 16)

    for jj in range(2):
        for k in range(RPT // 832):
            pltpu.sync_copy(zb_v, spm.at[pl.ds(s * RPT + k * 832, 832)])
        plsc.subcore_barrier()

        def win(w, _):
            base = wid * EPT + w * W
            pltpu.sync_copy(src_h.at[pl.ds(base, W)], src_v)
            pltpu.sync_copy(dst_h.at[pl.ds(base, W)], dst_v)
            pltpu.sync_copy(exf_h.at[pl.ds(base * 4, W * 4)],
                            ex_v.at[pl.ds(0, W * 4)])
            pltpu.async_copy(xl_h.at[src_v], xl_v, sem).wait()
            pltpu.async_copy(dm_h.at[dst_v], den_v, sem).wait()

            def edge(e, _):
                exv = ex_v[pl.ds(e * 4, 16)]
                dv = den_v[e, pl.ds(0, 16)]
                wv = exv / (dv + 1e-16)
                acc = xl_v[e, pl.ds(16 * jj, 16)] * wv[0]
                for h in range(1, 4):
                    acc = acc + xl_v[e, pl.ds(32 * h + 16 * jj, 16)] * wv[h]
                sc_v[e, pl.ds(0, 16)] = acc
                return 0

            lax.fori_loop(0, W, edge, 0)
            pltpu.sync_copy(sc_v, spm.at[dst_v], add=True)
            return 0

        lax.fori_loop(0, EPT // W, win, 0)
        plsc.subcore_barrier()
        pltpu.sync_copy(spm.at[pl.ds(s * RPT, RPT)],
                        outs[jj].at[c, pl.ds(s * RPT, RPT)])


@functools.partial(
    pl.kernel,
    out_type=tuple(jax.ShapeDtypeStruct((NC, NP, 16), jnp.float32)
                   for _ in range(2)),
    mesh=_MESH,
    compiler_params=_SC_PARAMS,
    scratch_types=[
        pltpu.VMEM((W,), jnp.int32),
        pltpu.VMEM((W,), jnp.int32),
        pltpu.VMEM((W * 4 + 16,), jnp.float32),
        pltpu.VMEM((W, 16), jnp.float32),
        pltpu.VMEM((W, 128), jnp.float32),
        pltpu.VMEM((W, 16), jnp.float32),
        pltpu.VMEM((832, 16), jnp.float32),
        pltpu.MemorySpace.VMEM_SHARED((NP, 16), jnp.float32),
        pltpu.SemaphoreType.DMA,
    ],
)
def _p3c(*args):
    _p3c_body(args)


def _make_p3(NSL, CPH):
    @functools.partial(
        pl.kernel,
        out_type=tuple(jax.ShapeDtypeStruct((NC, NP, 16), jnp.float32)
                       for _ in range(NSL)),
        mesh=_MESH,
        compiler_params=_SC_PARAMS,
        scratch_types=[
            pltpu.VMEM((W,), jnp.int32),
            pltpu.VMEM((W,), jnp.int32),
            pltpu.VMEM((W * 4 + 16,), jnp.float32),
            pltpu.VMEM((W, 16), jnp.float32),
            pltpu.VMEM((W, 16), jnp.float32),
            pltpu.VMEM((832, 16), jnp.float32),
            pltpu.MemorySpace.VMEM_SHARED((NP, 16), jnp.float32),
            pltpu.SemaphoreType.DMA,
        ],
    )
    def _p3(*args):
        _p3_body(NSL, CPH, args)
    return _p3


def _pool_body(args):
    parts = args[:2]
    b_h, b2_h, g_h = args[2:5]
    b_v, b2_v, bufs, hrow_v, zb_v, spm, sem = args[5:]
    c, s, wid = _ids()
    pltpu.sync_copy(b2_h, b2_v)
    _zero_rows(zb_v, 32)
    pltpu.sync_copy(zb_v.at[pl.ds(0, GP // NS)], spm.at[pl.ds(s * (GP // NS), GP // NS)])
    plsc.subcore_barrier()

    def win(w, _):
        base = wid * RPW + w * W
        pltpu.sync_copy(b_h.at[pl.ds(base, W)], b_v)
        for t in range(2):
            for cc in range(NC):
                pltpu.sync_copy(parts[t].at[cc, pl.ds(base, W)], bufs[t * 2 + cc])

        def row(r, _):
            for jj in range(2):
                acc = bufs[2 * jj][r, pl.ds(0, 16)] + bufs[2 * jj + 1][r, pl.ds(0, 16)]
                hrow_v[r, pl.ds(jj * 16, 16)] = acc * 0.25 + b2_v[pl.ds(jj * 16, 16)]
            return 0

        lax.fori_loop(0, W, row, 0)
        pltpu.sync_copy(hrow_v, spm.at[b_v], add=True)
        return 0

    lax.fori_loop(0, RPW // W, win, 0)
    plsc.subcore_barrier()
    pltpu.sync_copy(spm.at[pl.ds(s * (GP // NS), GP // NS)],
                    g_h.at[c, pl.ds(s * (GP // NS), GP // NS)])


@functools.partial(
    pl.kernel,
    out_type=jax.ShapeDtypeStruct((NC, GP, 32), jnp.float32),
    mesh=_MESH,
    compiler_params=_SC_PARAMS,
    scratch_types=[
        pltpu.VMEM((W,), jnp.int32),
        pltpu.VMEM((32,), jnp.float32),
        [pltpu.VMEM((W, 16), jnp.float32)] * 4,
        pltpu.VMEM((W, 32), jnp.float32),
        pltpu.VMEM((832, 32), jnp.float32),
        pltpu.MemorySpace.VMEM_SHARED((GP, 32), jnp.float32),
        pltpu.SemaphoreType.DMA,
    ],
)
def _pool(*args):
    _pool_body(args)


_R = 512
_GRID = NP // _R


def _tc1_body(x_ref, wl_ref, wr_ref, s0_ref, s1_ref,
              xl_ref, t0_ref, t1_ref, t2_ref, t3_ref, xr_ref, la_ref):
    xb = x_ref[...]
    xl = jnp.dot(xb, wl_ref[...], preferred_element_type=jnp.float32)
    xl_ref[...] = xl
    for h, t_ref in enumerate((t0_ref, t1_ref, t2_ref, t3_ref)):
        t_ref[...] = xl[:, 16 * h:16 * h + 16]
    xr_ref[...] = jnp.dot(xb, wr_ref[...], preferred_element_type=jnp.float32)
    ssum = s0_ref[...] + s1_ref[...]
    la_ref[...] = ssum / jnp.clip(ssum[:, 4:5], 1.0, None)


def _tc1(x_p, Wl1, Wr1, sums):
    return pl.pallas_call(
        _tc1_body,
        grid=(_GRID,),
        in_specs=[
            pl.BlockSpec((_R, 32), lambda i: (i, 0)),
            pl.BlockSpec((32, 64), lambda i: (0, 0)),
            pl.BlockSpec((32, 64), lambda i: (0, 0)),
            pl.BlockSpec((_R, 16), lambda i: (i, 0)),
            pl.BlockSpec((_R, 16), lambda i: (i, 0)),
        ],
        out_specs=[pl.BlockSpec((_R, 64), lambda i: (i, 0))]
        + [pl.BlockSpec((_R, 16), lambda i: (i, 0))] * 4
        + [pl.BlockSpec((_R, 64), lambda i: (i, 0)),
           pl.BlockSpec((_R, 16), lambda i: (i, 0))],
        out_shape=[jax.ShapeDtypeStruct((NP, 64), jnp.float32)]
        + [jax.ShapeDtypeStruct((NP, 16), jnp.float32)] * 4
        + [jax.ShapeDtypeStruct((NP, 64), jnp.float32),
           jax.ShapeDtypeStruct((NP, 16), jnp.float32)],
    )(x_p, Wl1, Wr1, sums[0], sums[1])


def _tc2_body(p0_ref, p1_ref, p2_ref, p3_ref, d_ref, b1_ref, wl_ref, wr_ref,
              xl_ref, xr_ref):
    d = d_ref[0] + d_ref[1]
    segs = [(p_ref[0] + p_ref[1]) / (d[:, h:h + 1] + 1e-16)
            for h, p_ref in enumerate((p0_ref, p1_ref, p2_ref, p3_ref))]
    h = jnp.maximum(jnp.concatenate(segs, axis=1) + b1_ref[...], 0.0)
    xl_ref[...] = jnp.dot(h, wl_ref[...], preferred_element_type=jnp.float32)
    xr_ref[...] = jnp.dot(h, wr_ref[...], preferred_element_type=jnp.float32)


def _tc2(o1, den1, b1, Wl2, Wr2):
    return pl.pallas_call(
        _tc2_body,
        grid=(_GRID,),
        in_specs=[pl.BlockSpec((2, _R, 16), lambda i: (0, i, 0))] * 5
        + [
            pl.BlockSpec((1, 64), lambda i: (0, 0)),
            pl.BlockSpec((64, 128), lambda i: (0, 0)),
            pl.BlockSpec((64, 128), lambda i: (0, 0)),
        ],
        out_specs=[pl.BlockSpec((_R, 128), lambda i: (i, 0))] * 2,
        out_shape=[jax.ShapeDtypeStruct((NP, 128), jnp.float32)] * 2,
    )(o1[0], o1[1], o1[2], o1[3], den1, b1.reshape(1, 64), Wl2, Wr2)


def _merge_body(a_ref, b_ref, o_ref):
    o_ref[...] = a_ref[...] + b_ref[...]


def _merge16(den):
    return pl.pallas_call(
        _merge_body,
        grid=(_GRID,),
        in_specs=[pl.BlockSpec((_R, 16), lambda i: (i, 0)),
                  pl.BlockSpec((_R, 16), lambda i: (i, 0))],
        out_specs=pl.BlockSpec((_R, 16), lambda i: (i, 0)),
        out_shape=jax.ShapeDtypeStruct((NP, 16), jnp.float32),
    )(den[0], den[1])


def _tc4_body(g_ref, w3_ref, b3_ref, gm_ref, bt_ref, w4_ref, b4_ref, o_ref):
    g = g_ref[0, :G, :] + g_ref[1, :G, :]
    h = jnp.maximum(jnp.dot(g, w3_ref[...], preferred_element_type=jnp.float32)
                    + b3_ref[...], 0.0)
    mu = jnp.mean(h, axis=-1, keepdims=True)
    var = jnp.mean((h - mu) ** 2, axis=-1, keepdims=True)
    hn = (h - mu) * jax.lax.rsqrt(var + 1e-5) * gm_ref[...] + bt_ref[...]
    o_ref[...] = jnp.dot(hn, w4_ref[...], preferred_element_type=jnp.float32) + b4_ref[...]


def _tc4(gacc, W3, b3, gamma, beta, W4, b4):
    return pl.pallas_call(
        _tc4_body,
        out_shape=jax.ShapeDtypeStruct((G, 64), jnp.float32),
    )(gacc, W3, b3.reshape(1, 128), gamma.reshape(1, 128),
      beta.reshape(1, 128), W4, b4.reshape(1, 64))


_P1_L1 = _make_p1(64, 16)
_P1_L2 = _make_p1(128, 32)
_P3_L1 = _make_p3(4, 1)


def kernel(x, edge_index, edge_attr, batch, Wl1, Wr1, We1, att1, b1,
           Wl2, Wr2, We2, att2, b2, W3, b3, gamma, beta, W4, b4):
    src0 = edge_index[0]
    dst0 = edge_index[1]

    pad0 = E0P - E
    dst0_p = jnp.concatenate([dst0, N + (jnp.arange(pad0, dtype=jnp.int32) % 16)])
    ea0_f = jnp.concatenate(
        [edge_attr, jnp.zeros((pad0, 4), jnp.float32)]).reshape(-1)

    x_p = jnp.pad(x, ((0, NP - N), (0, 0)))

    sums = _p0(dst0_p, ea0_f)
    xl1, t0, t1, t2, t3, xr1, la = _tc1(x_p, Wl1, Wr1, sums)

    loop_attr = la[:N, :4]
    pad1 = EP - E - N
    loop_idx = jnp.arange(N, dtype=jnp.int32)
    pad_idx = N + (jnp.arange(pad1, dtype=jnp.int32) % 16)
    src_p = jnp.concatenate([src0, loop_idx, pad_idx])
    dst_p = jnp.concatenate([dst0, loop_idx, pad_idx])
    ea_f = jnp.concatenate(
        [edge_attr, loop_attr, jnp.zeros((pad1, 4), jnp.float32)]).reshape(-1)

    ex1, den1 = _P1_L1(xl1, xr1, src_p, dst_p, ea_f, We1, att1)
    o1 = _P3_L1(t0, t1, t2, t3, src_p, dst_p, ex1)

    xl2, xr2 = _tc2(o1, den1, b1, Wl2, Wr2)

    ex2, den2 = _P1_L2(xl2, xr2, src_p, dst_p, ea_f, We2, att2)
    den2m = _merge16(den2)
    o2 = _p3c(xl2, den2m, src_p, dst_p, ex2)

    batch_p = jnp.concatenate(
        [batch, G + (jnp.arange(NP - N, dtype=jnp.int32) % 32)])
    gacc = _pool(*o2, batch_p, b2)
    return _tc4(gacc, W3, b3, gamma, beta, W4, b4)


# speedup vs baseline: 22.3936x; 1.2841x over previous
import functools

import jax
import jax.numpy as jnp
from jax import lax
from jax.experimental import pallas as pl
from jax.experimental.pallas import tpu as pltpu
from jax.experimental.pallas import tpu_sc as plsc

N = 50000
E = 800000
G = 512
H = 4

NC = 2
NS = 16
NW = NC * NS

W = 128
NP = 53248
RPT = NP // NS
RPW = NP // NW

E0PT = 25088
E0P = E0PT * NW
EPT = 26624
EP = EPT * NW

GP = 544

_MESH = plsc.VectorSubcoreMesh(core_axis_name="c", subcore_axis_name="s",
                               num_cores=NC, num_subcores=NS)
_SC_PARAMS = pltpu.CompilerParams(use_tc_tiling_on_sc=False,
                                  needs_layout_passes=False)


def _ids():
    c = lax.axis_index("c")
    s = lax.axis_index("s")
    return c, s, s * NC + c


def _zero_rows(zb, n_lanes):
    z = jnp.zeros((16,), jnp.float32)
    def body(r, _):
        for jj in range(n_lanes // 16):
            zb[r, pl.ds(jj * 16, 16)] = z
        return 0
    lax.fori_loop(0, zb.shape[0], body, 0)


def _zero_spmem(zb, spm, s):
    _zero_rows(zb, zb.shape[1])
    for k in range(RPT // 832):
        pltpu.sync_copy(zb, spm.at[pl.ds(s * RPT + k * 832, 832)])


def _p0_body(dst_h, eaf_h, sums_h, dst_v, ea_v, row_v, zb_v, spm, sem, sem2):
    c, s, wid = _ids()
    _zero_spmem(zb_v, spm, s)
    plsc.subcore_barrier()

    iota = lax.iota(jnp.int32, 16)

    def win(w, _):
        base = wid * E0PT + w * W
        d1 = pltpu.async_copy(dst_h.at[pl.ds(base, W)], dst_v, sem)
        d2 = pltpu.async_copy(eaf_h.at[pl.ds(base * 4, W * 4)],
                              ea_v.at[pl.ds(0, W * 4)], sem2)
        d1.wait()
        d2.wait()

        def edge4(i, _):
            for u in range(4):
                e = i * 4 + u
                av = ea_v[pl.ds(e * 4, 16)]
                row = jnp.where(iota < 4, av, 0.0)
                row = jnp.where(iota == 4, 1.0, row)
                row_v[e, pl.ds(0, 16)] = row
            return 0

        lax.fori_loop(0, W // 4, edge4, 0)
        pltpu.sync_copy(row_v, spm.at[dst_v], add=True)
        return 0

    lax.fori_loop(0, E0PT // W, win, 0)
    plsc.subcore_barrier()
    pltpu.sync_copy(spm.at[pl.ds(s * RPT, RPT)],
                    sums_h.at[c, pl.ds(s * RPT, RPT)])


@functools.partial(
    pl.kernel,
    out_type=jax.ShapeDtypeStruct((NC, NP, 16), jnp.float32),
    mesh=_MESH,
    compiler_params=_SC_PARAMS,
    scratch_types=[
        pltpu.VMEM((W,), jnp.int32),
        pltpu.VMEM((W * 4 + 16,), jnp.float32),
        pltpu.VMEM((W, 16), jnp.float32),
        pltpu.VMEM((832, 16), jnp.float32),
        pltpu.MemorySpace.VMEM_SHARED((NP, 16), jnp.float32),
        pltpu.SemaphoreType.DMA,
        pltpu.SemaphoreType.DMA,
    ],
)
def _p0(dst_h, eaf_h, sums_h, *rest):
    _p0_body(dst_h, eaf_h, sums_h, *rest)


def _p1_body(HC, C, xl_h, xr_h, src_h, dst_h, eaf_h, we_h, att_h,
             ex_h, den_h, src_v, dst_v, ea_v, xl_v, xr_v, ex_v,
             exb_v, we_v, att_v, zb_v, spm, sem, sem2, sem3):
    c, s, wid = _ids()
    C16 = C // 16
    _zero_spmem(zb_v, spm, s)
    pltpu.sync_copy(we_h, we_v)
    pltpu.sync_copy(att_h, att_v)
    plsc.subcore_barrier()

    iota = lax.iota(jnp.int32, 16)

    wes = [[we_v[k, pl.ds(j * 16, 16)] for j in range(HC // 16)] for k in range(4)]
    atts = [att_v[h, pl.ds(jj * 16, 16)] for h in range(H) for jj in range(C16)]

    def win(w, _):
        base = wid * EPT + w * W
        d1 = pltpu.async_copy(src_h.at[pl.ds(base, W)], src_v, sem)
        d2 = pltpu.async_copy(dst_h.at[pl.ds(base, W)], dst_v, sem2)
        d3 = pltpu.async_copy(eaf_h.at[pl.ds(base * 4, W * 4)],
                              ea_v.at[pl.ds(0, W * 4)], sem3)
        d1.wait()
        d2.wait()
        g1 = pltpu.async_copy(xl_h.at[src_v], xl_v, sem)
        g2 = pltpu.async_copy(xr_h.at[dst_v], xr_v, sem2)
        d3.wait()
        g1.wait()
        g2.wait()

        def edge4(i, _):
            for u in range(4):
                e = i * 4 + u
                av = ea_v[pl.ds(e * 4, 16)]
                a0, a1, a2, a3 = av[0], av[1], av[2], av[3]
                sv = jnp.zeros((16,), jnp.float32)
                for h in range(H):
                    sh = jnp.float32(0.0)
                    for jj in range(C16):
                        j = h * C16 + jj
                        ef = (a0 * wes[0][j] + a1 * wes[1][j]
                              + a2 * wes[2][j] + a3 * wes[3][j])
                        m = xl_v[e, pl.ds(j * 16, 16)] + xr_v[e, pl.ds(j * 16, 16)] + ef
                        t = jnp.maximum(m, 0.2 * m)
                        sh = sh + jnp.sum(atts[h * C16 + jj] * t)
                    sv = jnp.where(iota == h, sh, sv)
                ev = jnp.exp(sv)
                exb_v[e, pl.ds(0, 16)] = jnp.where(iota < 4, ev, 0.0)
                ex_v[pl.ds(e * 4, 16)] = ev
            return 0

        lax.fori_loop(0, W // 4, edge4, 0)
        pltpu.sync_copy(ex_v.at[pl.ds(0, W * 4)], ex_h.at[pl.ds(base * 4, W * 4)])
        pltpu.sync_copy(exb_v, spm.at[dst_v], add=True)
        return 0

    lax.fori_loop(0, EPT // W, win, 0)
    plsc.subcore_barrier()
    pltpu.sync_copy(spm.at[pl.ds(s * RPT, RPT)],
                    den_h.at[c, pl.ds(s * RPT, RPT)])


def _make_p1(HC, C):
    @functools.partial(
        pl.kernel,
        out_type=(jax.ShapeDtypeStruct((EP * 4,), jnp.float32),
                  jax.ShapeDtypeStruct((NC, NP, 16), jnp.float32)),
        mesh=_MESH,
        compiler_params=_SC_PARAMS,
        scratch_types=[
            pltpu.VMEM((W,), jnp.int32),
            pltpu.VMEM((W,), jnp.int32),
            pltpu.VMEM((W * 4 + 16,), jnp.float32),
            pltpu.VMEM((W, HC), jnp.float32),
            pltpu.VMEM((W, HC), jnp.float32),
            pltpu.VMEM((W * 4 + 16,), jnp.float32),
            pltpu.VMEM((W, 16), jnp.float32),
            pltpu.VMEM((4, HC), jnp.float32),
            pltpu.VMEM((4, C), jnp.float32),
            pltpu.VMEM((832, 16), jnp.float32),
            pltpu.MemorySpace.VMEM_SHARED((NP, 16), jnp.float32),
            pltpu.SemaphoreType.DMA,
            pltpu.SemaphoreType.DMA,
            pltpu.SemaphoreType.DMA,
        ],
    )
    def _p1(*args):
        _p1_body(HC, C, *args)
    return _p1


def _p3_body(NSL, CPH, args):
    tabs = args[:NSL]
    src_h, dst_h, exf_h = args[NSL:NSL + 3]
    outs = args[NSL + 3:2 * NSL + 3]
    src_v, dst_v, ex_v, xls_v, sc_v, zb_v, spm, sem, sem2, sem3 = args[2 * NSL + 3:]
    c, s, wid = _ids()
    _zero_rows(zb_v, 16)

    for sl in range(NSL):
        head = sl // CPH
        for k in range(RPT // 832):
            pltpu.sync_copy(zb_v, spm.at[pl.ds(s * RPT + k * 832, 832)])
        plsc.subcore_barrier()

        def win(w, _):
            base = wid * EPT + w * W
            d1 = pltpu.async_copy(src_h.at[pl.ds(base, W)], src_v, sem)
            d2 = pltpu.async_copy(dst_h.at[pl.ds(base, W)], dst_v, sem2)
            d3 = pltpu.async_copy(exf_h.at[pl.ds(base * 4, W * 4)],
                                  ex_v.at[pl.ds(0, W * 4)], sem3)
            d1.wait()
            g1 = pltpu.async_copy(tabs[sl].at[src_v], xls_v, sem)
            d2.wait()
            d3.wait()
            g1.wait()

            def edge4(i, _):
                for u in range(4):
                    e = i * 4 + u
                    exv = ex_v[pl.ds(e * 4, 16)]
                    sc_v[e, pl.ds(0, 16)] = xls_v[e, pl.ds(0, 16)] * exv[head]
                return 0

            lax.fori_loop(0, W // 4, edge4, 0)
            pltpu.sync_copy(sc_v, spm.at[dst_v], add=True)
            return 0

        lax.fori_loop(0, EPT // W, win, 0)
        plsc.subcore_barrier()
        pltpu.sync_copy(spm.at[pl.ds(s * RPT, RPT)],
                        outs[sl].at[c, pl.ds(s * RPT, RPT)])


def _p3c_body(args):
    xl_h, dm_h, src_h, dst_h, exf_h = args[:5]
    outs = args[5:7]
    (src_v, dst_v, ex_v, den_v, xl_v, sc_v, zb_v, spm,
     sem, sem2, sem3) = args[7:]
    c, s, wid = _ids()
    _zero_rows(zb_v, 16)

    for jj in range(2):
        for k in range(RPT // 832):
            pltpu.sync_copy(zb_v, spm.at[pl.ds(s * RPT + k * 832, 832)])
        plsc.subcore_barrier()

        def win(w, _):
            base = wid * EPT + w * W
            d1 = pltpu.async_copy(src_h.at[pl.ds(base, W)], src_v, sem)
            d2 = pltpu.async_copy(dst_h.at[pl.ds(base, W)], dst_v, sem2)
            d3 = pltpu.async_copy(exf_h.at[pl.ds(base * 4, W * 4)],
                                  ex_v.at[pl.ds(0, W * 4)], sem3)
            d1.wait()
            g1 = pltpu.async_copy(xl_h.at[src_v], xl_v, sem)
            d2.wait()
            g2 = pltpu.async_copy(dm_h.at[dst_v], den_v, sem2)
            d3.wait()
            g1.wait()
            g2.wait()

            def edge4(i, _):
                for u in range(4):
                    e = i * 4 + u
                    exv = ex_v[pl.ds(e * 4, 16)]
                    dv = den_v[e, pl.ds(0, 16)]
                    wv = exv / (dv + 1e-16)
                    acc = xl_v[e, pl.ds(16 * jj, 16)] * wv[0]
                    for h in range(1, 4):
                        acc = acc + xl_v[e, pl.ds(32 * h + 16 * jj, 16)] * wv[h]
                    sc_v[e, pl.ds(0, 16)] = acc
                return 0

            lax.fori_loop(0, W // 4, edge4, 0)
            pltpu.sync_copy(sc_v, spm.at[dst_v], add=True)
            return 0

        lax.fori_loop(0, EPT // W, win, 0)
        plsc.subcore_barrier()
        pltpu.sync_copy(spm.at[pl.ds(s * RPT, RPT)],
                        outs[jj].at[c, pl.ds(s * RPT, RPT)])


@functools.partial(
    pl.kernel,
    out_type=tuple(jax.ShapeDtypeStruct((NC, NP, 16), jnp.float32)
                   for _ in range(2)),
    mesh=_MESH,
    compiler_params=_SC_PARAMS,
    scratch_types=[
        pltpu.VMEM((W,), jnp.int32),
        pltpu.VMEM((W,), jnp.int32),
        pltpu.VMEM((W * 4 + 16,), jnp.float32),
        pltpu.VMEM((W, 16), jnp.float32),
        pltpu.VMEM((W, 128), jnp.float32),
        pltpu.VMEM((W, 16), jnp.float32),
        pltpu.VMEM((832, 16), jnp.float32),
        pltpu.MemorySpace.VMEM_SHARED((NP, 16), jnp.float32),
        pltpu.SemaphoreType.DMA,
        pltpu.SemaphoreType.DMA,
        pltpu.SemaphoreType.DMA,
    ],
)
def _p3c(*args):
    _p3c_body(args)


def _make_p3(NSL, CPH):
    @functools.partial(
        pl.kernel,
        out_type=tuple(jax.ShapeDtypeStruct((NC, NP, 16), jnp.float32)
                       for _ in range(NSL)),
        mesh=_MESH,
        compiler_params=_SC_PARAMS,
        scratch_types=[
            pltpu.VMEM((W,), jnp.int32),
            pltpu.VMEM((W,), jnp.int32),
            pltpu.VMEM((W * 4 + 16,), jnp.float32),
            pltpu.VMEM((W, 16), jnp.float32),
            pltpu.VMEM((W, 16), jnp.float32),
            pltpu.VMEM((832, 16), jnp.float32),
            pltpu.MemorySpace.VMEM_SHARED((NP, 16), jnp.float32),
            pltpu.SemaphoreType.DMA,
            pltpu.SemaphoreType.DMA,
            pltpu.SemaphoreType.DMA,
        ],
    )
    def _p3(*args):
        _p3_body(NSL, CPH, args)
    return _p3


def _pool_body(args):
    parts = args[:2]
    b_h, b2_h, g_h = args[2:5]
    b_v, b2_v, bufs, hrow_v, zb_v, spm, sem = args[5:]
    c, s, wid = _ids()
    pltpu.sync_copy(b2_h, b2_v)
    _zero_rows(zb_v, 32)
    pltpu.sync_copy(zb_v.at[pl.ds(0, GP // NS)], spm.at[pl.ds(s * (GP // NS), GP // NS)])
    plsc.subcore_barrier()

    def win(w, _):
        base = wid * RPW + w * W
        pltpu.sync_copy(b_h.at[pl.ds(base, W)], b_v)
        for t in range(2):
            for cc in range(NC):
                pltpu.sync_copy(parts[t].at[cc, pl.ds(base, W)], bufs[t * 2 + cc])

        def row(r, _):
            for jj in range(2):
                acc = bufs[2 * jj][r, pl.ds(0, 16)] + bufs[2 * jj + 1][r, pl.ds(0, 16)]
                hrow_v[r, pl.ds(jj * 16, 16)] = acc * 0.25 + b2_v[pl.ds(jj * 16, 16)]
            return 0

        lax.fori_loop(0, W, row, 0)
        pltpu.sync_copy(hrow_v, spm.at[b_v], add=True)
        return 0

    lax.fori_loop(0, RPW // W, win, 0)
    plsc.subcore_barrier()
    pltpu.sync_copy(spm.at[pl.ds(s * (GP // NS), GP // NS)],
                    g_h.at[c, pl.ds(s * (GP // NS), GP // NS)])


@functools.partial(
    pl.kernel,
    out_type=jax.ShapeDtypeStruct((NC, GP, 32), jnp.float32),
    mesh=_MESH,
    compiler_params=_SC_PARAMS,
    scratch_types=[
        pltpu.VMEM((W,), jnp.int32),
        pltpu.VMEM((32,), jnp.float32),
        [pltpu.VMEM((W, 16), jnp.float32)] * 4,
        pltpu.VMEM((W, 32), jnp.float32),
        pltpu.VMEM((832, 32), jnp.float32),
        pltpu.MemorySpace.VMEM_SHARED((GP, 32), jnp.float32),
        pltpu.SemaphoreType.DMA,
    ],
)
def _pool(*args):
    _pool_body(args)


_R = 512
_GRID = NP // _R


def _tc1_body(x_ref, wl_ref, wr_ref, s0_ref, s1_ref,
              xl_ref, t0_ref, t1_ref, t2_ref, t3_ref, xr_ref, la_ref):
    xb = x_ref[...]
    xl = jnp.dot(xb, wl_ref[...], preferred_element_type=jnp.float32)
    xl_ref[...] = xl
    for h, t_ref in enumerate((t0_ref, t1_ref, t2_ref, t3_ref)):
        t_ref[...] = xl[:, 16 * h:16 * h + 16]
    xr_ref[...] = jnp.dot(xb, wr_ref[...], preferred_element_type=jnp.float32)
    ssum = s0_ref[...] + s1_ref[...]
    la_ref[...] = ssum / jnp.clip(ssum[:, 4:5], 1.0, None)


def _tc1(x_p, Wl1, Wr1, sums):
    return pl.pallas_call(
        _tc1_body,
        grid=(_GRID,),
        in_specs=[
            pl.BlockSpec((_R, 32), lambda i: (i, 0)),
            pl.BlockSpec((32, 64), lambda i: (0, 0)),
            pl.BlockSpec((32, 64), lambda i: (0, 0)),
            pl.BlockSpec((_R, 16), lambda i: (i, 0)),
            pl.BlockSpec((_R, 16), lambda i: (i, 0)),
        ],
        out_specs=[pl.BlockSpec((_R, 64), lambda i: (i, 0))]
        + [pl.BlockSpec((_R, 16), lambda i: (i, 0))] * 4
        + [pl.BlockSpec((_R, 64), lambda i: (i, 0)),
           pl.BlockSpec((_R, 16), lambda i: (i, 0))],
        out_shape=[jax.ShapeDtypeStruct((NP, 64), jnp.float32)]
        + [jax.ShapeDtypeStruct((NP, 16), jnp.float32)] * 4
        + [jax.ShapeDtypeStruct((NP, 64), jnp.float32),
           jax.ShapeDtypeStruct((NP, 16), jnp.float32)],
    )(x_p, Wl1, Wr1, sums[0], sums[1])


def _tc2_body(p0_ref, p1_ref, p2_ref, p3_ref, d_ref, b1_ref, wl_ref, wr_ref,
              xl_ref, xr_ref):
    d = d_ref[0] + d_ref[1]
    segs = [(p_ref[0] + p_ref[1]) / (d[:, h:h + 1] + 1e-16)
            for h, p_ref in enumerate((p0_ref, p1_ref, p2_ref, p3_ref))]
    h = jnp.maximum(jnp.concatenate(segs, axis=1) + b1_ref[...], 0.0)
    xl_ref[...] = jnp.dot(h, wl_ref[...], preferred_element_type=jnp.float32)
    xr_ref[...] = jnp.dot(h, wr_ref[...], preferred_element_type=jnp.float32)


def _tc2(o1, den1, b1, Wl2, Wr2):
    return pl.pallas_call(
        _tc2_body,
        grid=(_GRID,),
        in_specs=[pl.BlockSpec((2, _R, 16), lambda i: (0, i, 0))] * 5
        + [
            pl.BlockSpec((1, 64), lambda i: (0, 0)),
            pl.BlockSpec((64, 128), lambda i: (0, 0)),
            pl.BlockSpec((64, 128), lambda i: (0, 0)),
        ],
        out_specs=[pl.BlockSpec((_R, 128), lambda i: (i, 0))] * 2,
        out_shape=[jax.ShapeDtypeStruct((NP, 128), jnp.float32)] * 2,
    )(o1[0], o1[1], o1[2], o1[3], den1, b1.reshape(1, 64), Wl2, Wr2)


def _merge_body(a_ref, b_ref, o_ref):
    o_ref[...] = a_ref[...] + b_ref[...]


def _merge16(den):
    return pl.pallas_call(
        _merge_body,
        grid=(_GRID,),
        in_specs=[pl.BlockSpec((_R, 16), lambda i: (i, 0)),
                  pl.BlockSpec((_R, 16), lambda i: (i, 0))],
        out_specs=pl.BlockSpec((_R, 16), lambda i: (i, 0)),
        out_shape=jax.ShapeDtypeStruct((NP, 16), jnp.float32),
    )(den[0], den[1])


def _tc4_body(g_ref, w3_ref, b3_ref, gm_ref, bt_ref, w4_ref, b4_ref, o_ref):
    g = g_ref[0, :G, :] + g_ref[1, :G, :]
    h = jnp.maximum(jnp.dot(g, w3_ref[...], preferred_element_type=jnp.float32)
                    + b3_ref[...], 0.0)
    mu = jnp.mean(h, axis=-1, keepdims=True)
    var = jnp.mean((h - mu) ** 2, axis=-1, keepdims=True)
    hn = (h - mu) * jax.lax.rsqrt(var + 1e-5) * gm_ref[...] + bt_ref[...]
    o_ref[...] = jnp.dot(hn, w4_ref[...], preferred_element_type=jnp.float32) + b4_ref[...]


def _tc4(gacc, W3, b3, gamma, beta, W4, b4):
    return pl.pallas_call(
        _tc4_body,
        out_shape=jax.ShapeDtypeStruct((G, 64), jnp.float32),
    )(gacc, W3, b3.reshape(1, 128), gamma.reshape(1, 128),
      beta.reshape(1, 128), W4, b4.reshape(1, 64))


_P1_L1 = _make_p1(64, 16)
_P1_L2 = _make_p1(128, 32)
_P3_L1 = _make_p3(4, 1)


def kernel(x, edge_index, edge_attr, batch, Wl1, Wr1, We1, att1, b1,
           Wl2, Wr2, We2, att2, b2, W3, b3, gamma, beta, W4, b4):
    src0 = edge_index[0]
    dst0 = edge_index[1]

    pad0 = E0P - E
    dst0_p = jnp.concatenate([dst0, N + (jnp.arange(pad0, dtype=jnp.int32) % 16)])
    ea0_f = jnp.concatenate(
        [edge_attr, jnp.zeros((pad0, 4), jnp.float32)]).reshape(-1)

    x_p = jnp.pad(x, ((0, NP - N), (0, 0)))

    sums = _p0(dst0_p, ea0_f)
    xl1, t0, t1, t2, t3, xr1, la = _tc1(x_p, Wl1, Wr1, sums)

    loop_attr = la[:N, :4]
    pad1 = EP - E - N
    loop_idx = jnp.arange(N, dtype=jnp.int32)
    pad_idx = N + (jnp.arange(pad1, dtype=jnp.int32) % 16)
    src_p = jnp.concatenate([src0, loop_idx, pad_idx])
    dst_p = jnp.concatenate([dst0, loop_idx, pad_idx])
    ea_f = jnp.concatenate(
        [edge_attr, loop_attr, jnp.zeros((pad1, 4), jnp.float32)]).reshape(-1)

    ex1, den1 = _P1_L1(xl1, xr1, src_p, dst_p, ea_f, We1, att1)
    o1 = _P3_L1(t0, t1, t2, t3, src_p, dst_p, ex1)

    xl2, xr2 = _tc2(o1, den1, b1, Wl2, Wr2)

    ex2, den2 = _P1_L2(xl2, xr2, src_p, dst_p, ea_f, We2, att2)
    den2m = _merge16(den2)
    o2 = _p3c(xl2, den2m, src_p, dst_p, ex2)

    batch_p = jnp.concatenate(
        [batch, G + (jnp.arange(NP - N, dtype=jnp.int32) % 32)])
    gacc = _pool(*o2, batch_p, b2)
    return _tc4(gacc, W3, b3, gamma, beta, W4, b4)


# speedup vs baseline: 22.5183x; 1.0056x over previous
import functools

import jax
import jax.numpy as jnp
from jax import lax
from jax.experimental import pallas as pl
from jax.experimental.pallas import tpu as pltpu
from jax.experimental.pallas import tpu_sc as plsc

N = 50000
E = 800000
G = 512
H = 4

NC = 2
NS = 16
NW = NC * NS

W = 128
NP = 53248
RPT = NP // NS
RPW = NP // NW

E0PT = 25088
E0P = E0PT * NW
EPT = 26624
EP = EPT * NW

GP = 544

_MESH = plsc.VectorSubcoreMesh(core_axis_name="c", subcore_axis_name="s",
                               num_cores=NC, num_subcores=NS)
_SC_PARAMS = pltpu.CompilerParams(use_tc_tiling_on_sc=False,
                                  needs_layout_passes=False)


def _ids():
    c = lax.axis_index("c")
    s = lax.axis_index("s")
    return c, s, s * NC + c


def _zero_rows(zb, n_lanes):
    z = jnp.zeros((16,), jnp.float32)
    def body(r, _):
        for jj in range(n_lanes // 16):
            zb[r, pl.ds(jj * 16, 16)] = z
        return 0
    lax.fori_loop(0, zb.shape[0], body, 0)


def _zero_spmem(zb, spm, s):
    _zero_rows(zb, zb.shape[1])
    for k in range(RPT // 832):
        pltpu.sync_copy(zb, spm.at[pl.ds(s * RPT + k * 832, 832)])


def _p0_body(dst_h, eaf_h, sums_h, dst_v, ea_v, row_v, zb_v, spm, sem, sem2):
    c, s, wid = _ids()
    _zero_spmem(zb_v, spm, s)
    plsc.subcore_barrier()

    iota = lax.iota(jnp.int32, 16)

    def win(w, _):
        base = wid * E0PT + w * W
        d1 = pltpu.async_copy(dst_h.at[pl.ds(base, W)], dst_v, sem)
        d2 = pltpu.async_copy(eaf_h.at[pl.ds(base * 4, W * 4)],
                              ea_v.at[pl.ds(0, W * 4)], sem2)
        d1.wait()
        d2.wait()

        def edge4(i, _):
            for u in range(4):
                e = i * 4 + u
                av = ea_v[pl.ds(e * 4, 16)]
                row = jnp.where(iota < 4, av, 0.0)
                row = jnp.where(iota == 4, 1.0, row)
                row_v[e, pl.ds(0, 16)] = row
            return 0

        lax.fori_loop(0, W // 4, edge4, 0)
        pltpu.sync_copy(row_v, spm.at[dst_v], add=True)
        return 0

    lax.fori_loop(0, E0PT // W, win, 0)
    plsc.subcore_barrier()
    pltpu.sync_copy(spm.at[pl.ds(s * RPT, RPT)],
                    sums_h.at[c, pl.ds(s * RPT, RPT)])


@functools.partial(
    pl.kernel,
    out_type=jax.ShapeDtypeStruct((NC, NP, 16), jnp.float32),
    mesh=_MESH,
    compiler_params=_SC_PARAMS,
    scratch_types=[
        pltpu.VMEM((W,), jnp.int32),
        pltpu.VMEM((W * 4 + 16,), jnp.float32),
        pltpu.VMEM((W, 16), jnp.float32),
        pltpu.VMEM((832, 16), jnp.float32),
        pltpu.MemorySpace.VMEM_SHARED((NP, 16), jnp.float32),
        pltpu.SemaphoreType.DMA,
        pltpu.SemaphoreType.DMA,
    ],
)
def _p0(dst_h, eaf_h, sums_h, *rest):
    _p0_body(dst_h, eaf_h, sums_h, *rest)


def _p1_body(HC, C, xl_h, xr_h, ef_h, src_h, dst_h, att_h,
             ex_h, den_h, src_v, dst_v, ef_v, xl_v, xr_v, ex_v,
             exb_v, att_v, zb_v, spm, sem, sem2, sem3):
    c, s, wid = _ids()
    C16 = C // 16
    _zero_spmem(zb_v, spm, s)
    pltpu.sync_copy(att_h, att_v)
    plsc.subcore_barrier()

    iota = lax.iota(jnp.int32, 16)
    atts = [att_v[h, pl.ds(jj * 16, 16)] for h in range(H) for jj in range(C16)]

    def win(w, _):
        base = wid * EPT + w * W
        d1 = pltpu.async_copy(src_h.at[pl.ds(base, W)], src_v, sem)
        d2 = pltpu.async_copy(dst_h.at[pl.ds(base, W)], dst_v, sem2)
        d3 = pltpu.async_copy(ef_h.at[pl.ds(base, W)], ef_v, sem3)
        d1.wait()
        d2.wait()
        g1 = pltpu.async_copy(xl_h.at[src_v], xl_v, sem)
        g2 = pltpu.async_copy(xr_h.at[dst_v], xr_v, sem2)
        d3.wait()
        g1.wait()
        g2.wait()

        def edge4(i, _):
            for u in range(4):
                e = i * 4 + u
                sv = jnp.zeros((16,), jnp.float32)
                for h in range(H):
                    sh = jnp.float32(0.0)
                    for jj in range(C16):
                        j = h * C16 + jj
                        m = (xl_v[e, pl.ds(j * 16, 16)]
                             + xr_v[e, pl.ds(j * 16, 16)]
                             + ef_v[e, pl.ds(j * 16, 16)])
                        t = jnp.maximum(m, 0.2 * m)
                        sh = sh + jnp.sum(atts[h * C16 + jj] * t)
                    sv = jnp.where(iota == h, sh, sv)
                ev = jnp.exp(sv)
                exb_v[e, pl.ds(0, 16)] = jnp.where(iota < 4, ev, 0.0)
                ex_v[pl.ds(e * 4, 16)] = ev
            return 0

        lax.fori_loop(0, W // 4, edge4, 0)
        pltpu.sync_copy(ex_v.at[pl.ds(0, W * 4)], ex_h.at[pl.ds(base * 4, W * 4)])
        pltpu.sync_copy(exb_v, spm.at[dst_v], add=True)
        return 0

    lax.fori_loop(0, EPT // W, win, 0)
    plsc.subcore_barrier()
    pltpu.sync_copy(spm.at[pl.ds(s * RPT, RPT)],
                    den_h.at[c, pl.ds(s * RPT, RPT)])


def _make_p1(HC, C):
    @functools.partial(
        pl.kernel,
        out_type=(jax.ShapeDtypeStruct((EP * 4,), jnp.float32),
                  jax.ShapeDtypeStruct((NC, NP, 16), jnp.float32)),
        mesh=_MESH,
        compiler_params=_SC_PARAMS,
        scratch_types=[
            pltpu.VMEM((W,), jnp.int32),
            pltpu.VMEM((W,), jnp.int32),
            pltpu.VMEM((W, HC), jnp.float32),
            pltpu.VMEM((W, HC), jnp.float32),
            pltpu.VMEM((W, HC), jnp.float32),
            pltpu.VMEM((W * 4 + 16,), jnp.float32),
            pltpu.VMEM((W, 16), jnp.float32),
            pltpu.VMEM((4, C), jnp.float32),
            pltpu.VMEM((832, 16), jnp.float32),
            pltpu.MemorySpace.VMEM_SHARED((NP, 16), jnp.float32),
            pltpu.SemaphoreType.DMA,
            pltpu.SemaphoreType.DMA,
            pltpu.SemaphoreType.DMA,
        ],
    )
    def _p1(*args):
        _p1_body(HC, C, *args)
    return _p1


def _p3_body(NSL, CPH, args):
    tabs = args[:NSL]
    src_h, dst_h, exf_h = args[NSL:NSL + 3]
    outs = args[NSL + 3:2 * NSL + 3]
    src_v, dst_v, ex_v, xls_v, sc_v, zb_v, spm, sem, sem2, sem3 = args[2 * NSL + 3:]
    c, s, wid = _ids()
    _zero_rows(zb_v, 16)

    for sl in range(NSL):
        head = sl // CPH
        for k in range(RPT // 832):
            pltpu.sync_copy(zb_v, spm.at[pl.ds(s * RPT + k * 832, 832)])
        plsc.subcore_barrier()

        def win(w, _):
            base = wid * EPT + w * W
            d1 = pltpu.async_copy(src_h.at[pl.ds(base, W)], src_v, sem)
            d2 = pltpu.async_copy(dst_h.at[pl.ds(base, W)], dst_v, sem2)
            d3 = pltpu.async_copy(exf_h.at[pl.ds(base * 4, W * 4)],
                                  ex_v.at[pl.ds(0, W * 4)], sem3)
            d1.wait()
            g1 = pltpu.async_copy(tabs[sl].at[src_v], xls_v, sem)
            d2.wait()
            d3.wait()
            g1.wait()

            def edge4(i, _):
                for u in range(4):
                    e = i * 4 + u
                    exv = ex_v[pl.ds(e * 4, 16)]
                    sc_v[e, pl.ds(0, 16)] = xls_v[e, pl.ds(0, 16)] * exv[head]
                return 0

            lax.fori_loop(0, W // 4, edge4, 0)
            pltpu.sync_copy(sc_v, spm.at[dst_v], add=True)
            return 0

        lax.fori_loop(0, EPT // W, win, 0)
        plsc.subcore_barrier()
        pltpu.sync_copy(spm.at[pl.ds(s * RPT, RPT)],
                        outs[sl].at[c, pl.ds(s * RPT, RPT)])


def _p3c_body(args):
    xl_h, dm_h, src_h, dst_h, exf_h = args[:5]
    outs = args[5:7]
    (src_v, dst_v, ex_v, den_v, xl_v, sc_v, zb_v, spm,
     sem, sem2, sem3) = args[7:]
    c, s, wid = _ids()
    _zero_rows(zb_v, 16)

    for jj in range(2):
        for k in range(RPT // 832):
            pltpu.sync_copy(zb_v, spm.at[pl.ds(s * RPT + k * 832, 832)])
        plsc.subcore_barrier()

        def win(w, _):
            base = wid * EPT + w * W
            d1 = pltpu.async_copy(src_h.at[pl.ds(base, W)], src_v, sem)
            d2 = pltpu.async_copy(dst_h.at[pl.ds(base, W)], dst_v, sem2)
            d3 = pltpu.async_copy(exf_h.at[pl.ds(base * 4, W * 4)],
                                  ex_v.at[pl.ds(0, W * 4)], sem3)
            d1.wait()
            g1 = pltpu.async_copy(xl_h.at[src_v], xl_v, sem)
            d2.wait()
            g2 = pltpu.async_copy(dm_h.at[dst_v], den_v, sem2)
            d3.wait()
            g1.wait()
            g2.wait()

            def edge4(i, _):
                for u in range(4):
                    e = i * 4 + u
                    exv = ex_v[pl.ds(e * 4, 16)]
                    dv = den_v[e, pl.ds(0, 16)]
                    wv = exv / (dv + 1e-16)
                    acc = xl_v[e, pl.ds(16 * jj, 16)] * wv[0]
                    for h in range(1, 4):
                        acc = acc + xl_v[e, pl.ds(32 * h + 16 * jj, 16)] * wv[h]
                    sc_v[e, pl.ds(0, 16)] = acc
                return 0

            lax.fori_loop(0, W // 4, edge4, 0)
            pltpu.sync_copy(sc_v, spm.at[dst_v], add=True)
            return 0

        lax.fori_loop(0, EPT // W, win, 0)
        plsc.subcore_barrier()
        pltpu.sync_copy(spm.at[pl.ds(s * RPT, RPT)],
                        outs[jj].at[c, pl.ds(s * RPT, RPT)])


@functools.partial(
    pl.kernel,
    out_type=tuple(jax.ShapeDtypeStruct((NC, NP, 16), jnp.float32)
                   for _ in range(2)),
    mesh=_MESH,
    compiler_params=_SC_PARAMS,
    scratch_types=[
        pltpu.VMEM((W,), jnp.int32),
        pltpu.VMEM((W,), jnp.int32),
        pltpu.VMEM((W * 4 + 16,), jnp.float32),
        pltpu.VMEM((W, 16), jnp.float32),
        pltpu.VMEM((W, 128), jnp.float32),
        pltpu.VMEM((W, 16), jnp.float32),
        pltpu.VMEM((832, 16), jnp.float32),
        pltpu.MemorySpace.VMEM_SHARED((NP, 16), jnp.float32),
        pltpu.SemaphoreType.DMA,
        pltpu.SemaphoreType.DMA,
        pltpu.SemaphoreType.DMA,
    ],
)
def _p3c(*args):
    _p3c_body(args)


def _make_p3(NSL, CPH):
    @functools.partial(
        pl.kernel,
        out_type=tuple(jax.ShapeDtypeStruct((NC, NP, 16), jnp.float32)
                       for _ in range(NSL)),
        mesh=_MESH,
        compiler_params=_SC_PARAMS,
        scratch_types=[
            pltpu.VMEM((W,), jnp.int32),
            pltpu.VMEM((W,), jnp.int32),
            pltpu.VMEM((W * 4 + 16,), jnp.float32),
            pltpu.VMEM((W, 16), jnp.float32),
            pltpu.VMEM((W, 16), jnp.float32),
            pltpu.VMEM((832, 16), jnp.float32),
            pltpu.MemorySpace.VMEM_SHARED((NP, 16), jnp.float32),
            pltpu.SemaphoreType.DMA,
            pltpu.SemaphoreType.DMA,
            pltpu.SemaphoreType.DMA,
        ],
    )
    def _p3(*args):
        _p3_body(NSL, CPH, args)
    return _p3


def _pool_body(args):
    parts = args[:2]
    b_h, b2_h, g_h = args[2:5]
    b_v, b2_v, bufs, hrow_v, zb_v, spm, sem = args[5:]
    c, s, wid = _ids()
    pltpu.sync_copy(b2_h, b2_v)
    _zero_rows(zb_v, 32)
    pltpu.sync_copy(zb_v.at[pl.ds(0, GP // NS)], spm.at[pl.ds(s * (GP // NS), GP // NS)])
    plsc.subcore_barrier()

    def win(w, _):
        base = wid * RPW + w * W
        pltpu.sync_copy(b_h.at[pl.ds(base, W)], b_v)
        for t in range(2):
            for cc in range(NC):
                pltpu.sync_copy(parts[t].at[cc, pl.ds(base, W)], bufs[t * 2 + cc])

        def row(r, _):
            for jj in range(2):
                acc = bufs[2 * jj][r, pl.ds(0, 16)] + bufs[2 * jj + 1][r, pl.ds(0, 16)]
                hrow_v[r, pl.ds(jj * 16, 16)] = acc * 0.25 + b2_v[pl.ds(jj * 16, 16)]
            return 0

        lax.fori_loop(0, W, row, 0)
        pltpu.sync_copy(hrow_v, spm.at[b_v], add=True)
        return 0

    lax.fori_loop(0, RPW // W, win, 0)
    plsc.subcore_barrier()
    pltpu.sync_copy(spm.at[pl.ds(s * (GP // NS), GP // NS)],
                    g_h.at[c, pl.ds(s * (GP // NS), GP // NS)])


@functools.partial(
    pl.kernel,
    out_type=jax.ShapeDtypeStruct((NC, GP, 32), jnp.float32),
    mesh=_MESH,
    compiler_params=_SC_PARAMS,
    scratch_types=[
        pltpu.VMEM((W,), jnp.int32),
        pltpu.VMEM((32,), jnp.float32),
        [pltpu.VMEM((W, 16), jnp.float32)] * 4,
        pltpu.VMEM((W, 32), jnp.float32),
        pltpu.VMEM((832, 32), jnp.float32),
        pltpu.MemorySpace.VMEM_SHARED((GP, 32), jnp.float32),
        pltpu.SemaphoreType.DMA,
    ],
)
def _pool(*args):
    _pool_body(args)


_R = 512
_GRID = NP // _R


def _tc1_body(x_ref, wl_ref, wr_ref, s0_ref, s1_ref,
              xl_ref, t0_ref, t1_ref, t2_ref, t3_ref, xr_ref, la_ref):
    xb = x_ref[...]
    xl = jnp.dot(xb, wl_ref[...], preferred_element_type=jnp.float32)
    xl_ref[...] = xl
    for h, t_ref in enumerate((t0_ref, t1_ref, t2_ref, t3_ref)):
        t_ref[...] = xl[:, 16 * h:16 * h + 16]
    xr_ref[...] = jnp.dot(xb, wr_ref[...], preferred_element_type=jnp.float32)
    ssum = s0_ref[...] + s1_ref[...]
    la_ref[...] = ssum / jnp.clip(ssum[:, 4:5], 1.0, None)


def _tc1(x_p, Wl1, Wr1, sums):
    return pl.pallas_call(
        _tc1_body,
        grid=(_GRID,),
        in_specs=[
            pl.BlockSpec((_R, 32), lambda i: (i, 0)),
            pl.BlockSpec((32, 64), lambda i: (0, 0)),
            pl.BlockSpec((32, 64), lambda i: (0, 0)),
            pl.BlockSpec((_R, 16), lambda i: (i, 0)),
            pl.BlockSpec((_R, 16), lambda i: (i, 0)),
        ],
        out_specs=[pl.BlockSpec((_R, 64), lambda i: (i, 0))]
        + [pl.BlockSpec((_R, 16), lambda i: (i, 0))] * 4
        + [pl.BlockSpec((_R, 64), lambda i: (i, 0)),
           pl.BlockSpec((_R, 16), lambda i: (i, 0))],
        out_shape=[jax.ShapeDtypeStruct((NP, 64), jnp.float32)]
        + [jax.ShapeDtypeStruct((NP, 16), jnp.float32)] * 4
        + [jax.ShapeDtypeStruct((NP, 64), jnp.float32),
           jax.ShapeDtypeStruct((NP, 16), jnp.float32)],
    )(x_p, Wl1, Wr1, sums[0], sums[1])


def _tc2_body(p0_ref, p1_ref, p2_ref, p3_ref, d_ref, b1_ref, wl_ref, wr_ref,
              xl_ref, xr_ref):
    d = d_ref[0] + d_ref[1]
    segs = [(p_ref[0] + p_ref[1]) / (d[:, h:h + 1] + 1e-16)
            for h, p_ref in enumerate((p0_ref, p1_ref, p2_ref, p3_ref))]
    h = jnp.maximum(jnp.concatenate(segs, axis=1) + b1_ref[...], 0.0)
    xl_ref[...] = jnp.dot(h, wl_ref[...], preferred_element_type=jnp.float32)
    xr_ref[...] = jnp.dot(h, wr_ref[...], preferred_element_type=jnp.float32)


def _tc2(o1, den1, b1, Wl2, Wr2):
    return pl.pallas_call(
        _tc2_body,
        grid=(_GRID,),
        in_specs=[pl.BlockSpec((2, _R, 16), lambda i: (0, i, 0))] * 5
        + [
            pl.BlockSpec((1, 64), lambda i: (0, 0)),
            pl.BlockSpec((64, 128), lambda i: (0, 0)),
            pl.BlockSpec((64, 128), lambda i: (0, 0)),
        ],
        out_specs=[pl.BlockSpec((_R, 128), lambda i: (i, 0))] * 2,
        out_shape=[jax.ShapeDtypeStruct((NP, 128), jnp.float32)] * 2,
    )(o1[0], o1[1], o1[2], o1[3], den1, b1.reshape(1, 64), Wl2, Wr2)


_EB = EP // _GRID


def _ef_body(ea_ref, we_ref, o_ref):
    o_ref[...] = jnp.dot(ea_ref[...], we_ref[...],
                         preferred_element_type=jnp.float32)


def _ef(ea2d, We):
    HC = We.shape[1]
    return pl.pallas_call(
        _ef_body,
        grid=(_GRID,),
        in_specs=[pl.BlockSpec((_EB, 4), lambda i: (i, 0)),
                  pl.BlockSpec((4, HC), lambda i: (0, 0))],
        out_specs=pl.BlockSpec((_EB, HC), lambda i: (i, 0)),
        out_shape=jax.ShapeDtypeStruct((EP, HC), jnp.float32),
    )(ea2d, We)


def _merge_body(a_ref, b_ref, o_ref):
    o_ref[...] = a_ref[...] + b_ref[...]


def _merge16(den):
    return pl.pallas_call(
        _merge_body,
        grid=(_GRID,),
        in_specs=[pl.BlockSpec((_R, 16), lambda i: (i, 0)),
                  pl.BlockSpec((_R, 16), lambda i: (i, 0))],
        out_specs=pl.BlockSpec((_R, 16), lambda i: (i, 0)),
        out_shape=jax.ShapeDtypeStruct((NP, 16), jnp.float32),
    )(den[0], den[1])


def _tc4_body(g_ref, w3_ref, b3_ref, gm_ref, bt_ref, w4_ref, b4_ref, o_ref):
    g = g_ref[0, :G, :] + g_ref[1, :G, :]
    h = jnp.maximum(jnp.dot(g, w3_ref[...], preferred_element_type=jnp.float32)
                    + b3_ref[...], 0.0)
    mu = jnp.mean(h, axis=-1, keepdims=True)
    var = jnp.mean((h - mu) ** 2, axis=-1, keepdims=True)
    hn = (h - mu) * jax.lax.rsqrt(var + 1e-5) * gm_ref[...] + bt_ref[...]
    o_ref[...] = jnp.dot(hn, w4_ref[...], preferred_element_type=jnp.float32) + b4_ref[...]


def _tc4(gacc, W3, b3, gamma, beta, W4, b4):
    return pl.pallas_call(
        _tc4_body,
        out_shape=jax.ShapeDtypeStruct((G, 64), jnp.float32),
    )(gacc, W3, b3.reshape(1, 128), gamma.reshape(1, 128),
      beta.reshape(1, 128), W4, b4.reshape(1, 64))


_P1_L1 = _make_p1(64, 16)
_P1_L2 = _make_p1(128, 32)
_P3_L1 = _make_p3(4, 1)


def kernel(x, edge_index, edge_attr, batch, Wl1, Wr1, We1, att1, b1,
           Wl2, Wr2, We2, att2, b2, W3, b3, gamma, beta, W4, b4):
    src0 = edge_index[0]
    dst0 = edge_index[1]

    pad0 = E0P - E
    dst0_p = jnp.concatenate([dst0, N + (jnp.arange(pad0, dtype=jnp.int32) % 16)])
    ea0_f = jnp.concatenate(
        [edge_attr, jnp.zeros((pad0, 4), jnp.float32)]).reshape(-1)

    x_p = jnp.pad(x, ((0, NP - N), (0, 0)))

    sums = _p0(dst0_p, ea0_f)
    xl1, t0, t1, t2, t3, xr1, la = _tc1(x_p, Wl1, Wr1, sums)

    loop_attr = la[:N, :4]
    pad1 = EP - E - N
    loop_idx = jnp.arange(N, dtype=jnp.int32)
    pad_idx = N + (jnp.arange(pad1, dtype=jnp.int32) % 16)
    src_p = jnp.concatenate([src0, loop_idx, pad_idx])
    dst_p = jnp.concatenate([dst0, loop_idx, pad_idx])
    ea_f = jnp.concatenate(
        [edge_attr, loop_attr, jnp.zeros((pad1, 4), jnp.float32)]).reshape(-1)

    ea2d = ea_f.reshape(EP, 4)
    ef1 = _ef(ea2d, We1)
    ef2 = _ef(ea2d, We2)
    ex1, den1 = _P1_L1(xl1, xr1, ef1, src_p, dst_p, att1)
    o1 = _P3_L1(t0, t1, t2, t3, src_p, dst_p, ex1)

    xl2, xr2 = _tc2(o1, den1, b1, Wl2, Wr2)

    ex2, den2 = _P1_L2(xl2, xr2, ef2, src_p, dst_p, att2)
    den2m = _merge16(den2)
    o2 = _p3c(xl2, den2m, src_p, dst_p, ex2)

    batch_p = jnp.concatenate(
        [batch, G + (jnp.arange(NP - N, dtype=jnp.int32) % 32)])
    gacc = _pool(*o2, batch_p, b2)
    return _tc4(gacc, W3, b3, gamma, beta, W4, b4)


# speedup vs baseline: 24.3381x; 1.0808x over previous
import functools

import jax
import jax.numpy as jnp
from jax import lax
from jax.experimental import pallas as pl
from jax.experimental.pallas import tpu as pltpu
from jax.experimental.pallas import tpu_sc as plsc

N = 50000
E = 800000
G = 512
H = 4

NC = 2
NS = 16
NW = NC * NS

W = 128
W2 = 256
NP = 53248
RPT = NP // NS
RPW = NP // NW

E0PT = 25088
E0P = E0PT * NW
EPT = 26624
EP = EPT * NW

GP = 544

_MESH = plsc.VectorSubcoreMesh(core_axis_name="c", subcore_axis_name="s",
                               num_cores=NC, num_subcores=NS)
_SC_PARAMS = pltpu.CompilerParams(use_tc_tiling_on_sc=False,
                                  needs_layout_passes=False)


def _ids():
    c = lax.axis_index("c")
    s = lax.axis_index("s")
    return c, s, s * NC + c


def _zero_rows(zb, n_lanes):
    z = jnp.zeros((16,), jnp.float32)
    def body(r, _):
        for jj in range(n_lanes // 16):
            zb[r, pl.ds(jj * 16, 16)] = z
        return 0
    lax.fori_loop(0, zb.shape[0], body, 0)


def _zero_spmem(zb, spm, s):
    _zero_rows(zb, zb.shape[1])
    for k in range(RPT // 832):
        pltpu.sync_copy(zb, spm.at[pl.ds(s * RPT + k * 832, 832)])


def _p0_body(dst_h, eaf_h, sums_h, dst_v, ea_v, row_v, zb_v, spm, sem, sem2):
    c, s, wid = _ids()
    _zero_spmem(zb_v, spm, s)
    plsc.subcore_barrier()

    iota = lax.iota(jnp.int32, 16)

    def win(w, _):
        base = wid * E0PT + w * W
        d1 = pltpu.async_copy(dst_h.at[pl.ds(base, W)], dst_v, sem)
        d2 = pltpu.async_copy(eaf_h.at[pl.ds(base * 4, W * 4)],
                              ea_v.at[pl.ds(0, W * 4)], sem2)
        d1.wait()
        d2.wait()

        def edge4(i, _):
            for u in range(4):
                e = i * 4 + u
                av = ea_v[pl.ds(e * 4, 16)]
                row = jnp.where(iota < 4, av, 0.0)
                row = jnp.where(iota == 4, 1.0, row)
                row_v[e, pl.ds(0, 16)] = row
            return 0

        lax.fori_loop(0, W // 4, edge4, 0)
        pltpu.sync_copy(row_v, spm.at[dst_v], add=True)
        return 0

    lax.fori_loop(0, E0PT // W, win, 0)
    plsc.subcore_barrier()
    pltpu.sync_copy(spm.at[pl.ds(s * RPT, RPT)],
                    sums_h.at[c, pl.ds(s * RPT, RPT)])


@functools.partial(
    pl.kernel,
    out_type=jax.ShapeDtypeStruct((NC, NP, 16), jnp.float32),
    mesh=_MESH,
    compiler_params=_SC_PARAMS,
    scratch_types=[
        pltpu.VMEM((W,), jnp.int32),
        pltpu.VMEM((W * 4 + 16,), jnp.float32),
        pltpu.VMEM((W, 16), jnp.float32),
        pltpu.VMEM((832, 16), jnp.float32),
        pltpu.MemorySpace.VMEM_SHARED((NP, 16), jnp.float32),
        pltpu.SemaphoreType.DMA,
        pltpu.SemaphoreType.DMA,
    ],
)
def _p0(dst_h, eaf_h, sums_h, *rest):
    _p0_body(dst_h, eaf_h, sums_h, *rest)


def _p1_body(HC, C, WW, xl_h, xr_h, ef_h, src_h, dst_h, att_h,
             ex_h, den_h, srcs, dsts, ef_v, xl_v, xr_v, ex_v,
             exb_v, att_v, zb_v, spm, sem, sem2, sem3):
    c, s, wid = _ids()
    NG = WW // W
    C16 = C // 16
    _zero_spmem(zb_v, spm, s)
    pltpu.sync_copy(att_h, att_v)
    plsc.subcore_barrier()

    iota = lax.iota(jnp.int32, 16)
    atts = [att_v[h, pl.ds(jj * 16, 16)] for h in range(H) for jj in range(C16)]

    def win(w, _):
        base = wid * EPT + w * WW
        d1 = [pltpu.async_copy(src_h.at[pl.ds(base + g * W, W)], srcs[g], sem)
              for g in range(NG)]
        d2 = [pltpu.async_copy(dst_h.at[pl.ds(base + g * W, W)], dsts[g], sem2)
              for g in range(NG)]
        d3 = pltpu.async_copy(ef_h.at[pl.ds(base, WW)], ef_v, sem3)
        for d in d1:
            d.wait()
        g1 = [pltpu.async_copy(xl_h.at[srcs[g]], xl_v.at[pl.ds(g * W, W)], sem)
              for g in range(NG)]
        for d in d2:
            d.wait()
        g2 = [pltpu.async_copy(xr_h.at[dsts[g]], xr_v.at[pl.ds(g * W, W)], sem2)
              for g in range(NG)]
        d3.wait()
        for d in g1 + g2:
            d.wait()

        def edge4(i, _):
            for u in range(4):
                e = i * 4 + u
                sv = jnp.zeros((16,), jnp.float32)
                for h in range(H):
                    sh = jnp.float32(0.0)
                    for jj in range(C16):
                        j = h * C16 + jj
                        m = (xl_v[e, pl.ds(j * 16, 16)]
                             + xr_v[e, pl.ds(j * 16, 16)]
                             + ef_v[e, pl.ds(j * 16, 16)])
                        t = jnp.maximum(m, 0.2 * m)
                        sh = sh + jnp.sum(atts[h * C16 + jj] * t)
                    sv = jnp.where(iota == h, sh, sv)
                ev = jnp.exp(sv)
                exb_v[e, pl.ds(0, 16)] = jnp.where(iota < 4, ev, 0.0)
                ex_v[pl.ds(e * 4, 16)] = ev
            return 0

        lax.fori_loop(0, WW // 4, edge4, 0)
        pltpu.sync_copy(ex_v.at[pl.ds(0, WW * 4)], ex_h.at[pl.ds(base * 4, WW * 4)])
        for g in range(NG):
            pltpu.sync_copy(exb_v.at[pl.ds(g * W, W)], spm.at[dsts[g]], add=True)
        return 0

    lax.fori_loop(0, EPT // WW, win, 0)
    plsc.subcore_barrier()
    pltpu.sync_copy(spm.at[pl.ds(s * RPT, RPT)],
                    den_h.at[c, pl.ds(s * RPT, RPT)])


def _make_p1(HC, C, WW):
    NG = WW // W

    @functools.partial(
        pl.kernel,
        out_type=(jax.ShapeDtypeStruct((EP * 4,), jnp.float32),
                  jax.ShapeDtypeStruct((NC, NP, 16), jnp.float32)),
        mesh=_MESH,
        compiler_params=_SC_PARAMS,
        scratch_types=[
            [pltpu.VMEM((W,), jnp.int32)] * NG,
            [pltpu.VMEM((W,), jnp.int32)] * NG,
            pltpu.VMEM((WW, HC), jnp.float32),
            pltpu.VMEM((WW, HC), jnp.float32),
            pltpu.VMEM((WW, HC), jnp.float32),
            pltpu.VMEM((WW * 4 + 16,), jnp.float32),
            pltpu.VMEM((WW, 16), jnp.float32),
            pltpu.VMEM((4, C), jnp.float32),
            pltpu.VMEM((832, 16), jnp.float32),
            pltpu.MemorySpace.VMEM_SHARED((NP, 16), jnp.float32),
            pltpu.SemaphoreType.DMA,
            pltpu.SemaphoreType.DMA,
            pltpu.SemaphoreType.DMA,
        ],
    )
    def _p1(*args):
        _p1_body(HC, C, WW, *args)
    return _p1


def _p3_body(NSL, CPH, args):
    tabs = args[:NSL]
    src_h, dst_h, exf_h = args[NSL:NSL + 3]
    outs = args[NSL + 3:2 * NSL + 3]
    (src_a, src_b, dst_a, dst_b, ex_v, xls_v, sc_v, zb_v, spm,
     sem, sem2, sem3) = args[2 * NSL + 3:]
    c, s, wid = _ids()
    _zero_rows(zb_v, 16)

    for sl in range(NSL):
        head = sl // CPH
        for k in range(RPT // 832):
            pltpu.sync_copy(zb_v, spm.at[pl.ds(s * RPT + k * 832, 832)])
        plsc.subcore_barrier()

        def win(w, _):
            base = wid * EPT + w * W2
            d1a = pltpu.async_copy(src_h.at[pl.ds(base, W)], src_a, sem)
            d1b = pltpu.async_copy(src_h.at[pl.ds(base + W, W)], src_b, sem)
            d2a = pltpu.async_copy(dst_h.at[pl.ds(base, W)], dst_a, sem2)
            d2b = pltpu.async_copy(dst_h.at[pl.ds(base + W, W)], dst_b, sem2)
            d3 = pltpu.async_copy(exf_h.at[pl.ds(base * 4, W2 * 4)],
                                  ex_v.at[pl.ds(0, W2 * 4)], sem3)
            d1a.wait()
            d1b.wait()
            g1a = pltpu.async_copy(tabs[sl].at[src_a], xls_v.at[pl.ds(0, W)], sem)
            g1b = pltpu.async_copy(tabs[sl].at[src_b], xls_v.at[pl.ds(W, W)], sem)
            d2a.wait()
            d2b.wait()
            d3.wait()
            g1a.wait()
            g1b.wait()

            def edge4(i, _):
                for u in range(4):
                    e = i * 4 + u
                    exv = ex_v[pl.ds(e * 4, 16)]
                    sc_v[e, pl.ds(0, 16)] = xls_v[e, pl.ds(0, 16)] * exv[head]
                return 0

            lax.fori_loop(0, W2 // 4, edge4, 0)
            pltpu.sync_copy(sc_v.at[pl.ds(0, W)], spm.at[dst_a], add=True)
            pltpu.sync_copy(sc_v.at[pl.ds(W, W)], spm.at[dst_b], add=True)
            return 0

        lax.fori_loop(0, EPT // W2, win, 0)
        plsc.subcore_barrier()
        pltpu.sync_copy(spm.at[pl.ds(s * RPT, RPT)],
                        outs[sl].at[c, pl.ds(s * RPT, RPT)])


def _p3c_body(args):
    xl_h, dm_h, src_h, dst_h, exf_h = args[:5]
    outs = args[5:7]
    (src_a, src_b, dst_a, dst_b, ex_v, den_v, xl_v, sc_v, zb_v, spm,
     sem, sem2, sem3) = args[7:]
    c, s, wid = _ids()
    _zero_rows(zb_v, 16)

    for jj in range(2):
        for k in range(RPT // 832):
            pltpu.sync_copy(zb_v, spm.at[pl.ds(s * RPT + k * 832, 832)])
        plsc.subcore_barrier()

        def win(w, _):
            base = wid * EPT + w * W2
            d1a = pltpu.async_copy(src_h.at[pl.ds(base, W)], src_a, sem)
            d1b = pltpu.async_copy(src_h.at[pl.ds(base + W, W)], src_b, sem)
            d2a = pltpu.async_copy(dst_h.at[pl.ds(base, W)], dst_a, sem2)
            d2b = pltpu.async_copy(dst_h.at[pl.ds(base + W, W)], dst_b, sem2)
            d3 = pltpu.async_copy(exf_h.at[pl.ds(base * 4, W2 * 4)],
                                  ex_v.at[pl.ds(0, W2 * 4)], sem3)
            d1a.wait()
            d1b.wait()
            g1a = pltpu.async_copy(xl_h.at[src_a], xl_v.at[pl.ds(0, W)], sem)
            g1b = pltpu.async_copy(xl_h.at[src_b], xl_v.at[pl.ds(W, W)], sem)
            d2a.wait()
            d2b.wait()
            g2a = pltpu.async_copy(dm_h.at[dst_a], den_v.at[pl.ds(0, W)], sem2)
            g2b = pltpu.async_copy(dm_h.at[dst_b], den_v.at[pl.ds(W, W)], sem2)
            d3.wait()
            g1a.wait()
            g1b.wait()
            g2a.wait()
            g2b.wait()

            def edge4(i, _):
                for u in range(4):
                    e = i * 4 + u
                    exv = ex_v[pl.ds(e * 4, 16)]
                    dv = den_v[e, pl.ds(0, 16)]
                    wv = exv / (dv + 1e-16)
                    acc = xl_v[e, pl.ds(16 * jj, 16)] * wv[0]
                    for h in range(1, 4):
                        acc = acc + xl_v[e, pl.ds(32 * h + 16 * jj, 16)] * wv[h]
                    sc_v[e, pl.ds(0, 16)] = acc
                return 0

            lax.fori_loop(0, W2 // 4, edge4, 0)
            pltpu.sync_copy(sc_v.at[pl.ds(0, W)], spm.at[dst_a], add=True)
            pltpu.sync_copy(sc_v.at[pl.ds(W, W)], spm.at[dst_b], add=True)
            return 0

        lax.fori_loop(0, EPT // W2, win, 0)
        plsc.subcore_barrier()
        pltpu.sync_copy(spm.at[pl.ds(s * RPT, RPT)],
                        outs[jj].at[c, pl.ds(s * RPT, RPT)])


@functools.partial(
    pl.kernel,
    out_type=tuple(jax.ShapeDtypeStruct((NC, NP, 16), jnp.float32)
                   for _ in range(2)),
    mesh=_MESH,
    compiler_params=_SC_PARAMS,
    scratch_types=[
        pltpu.VMEM((W,), jnp.int32),
        pltpu.VMEM((W,), jnp.int32),
        pltpu.VMEM((W,), jnp.int32),
        pltpu.VMEM((W,), jnp.int32),
        pltpu.VMEM((W2 * 4 + 16,), jnp.float32),
        pltpu.VMEM((W2, 16), jnp.float32),
        pltpu.VMEM((W2, 128), jnp.float32),
        pltpu.VMEM((W2, 16), jnp.float32),
        pltpu.VMEM((832, 16), jnp.float32),
        pltpu.MemorySpace.VMEM_SHARED((NP, 16), jnp.float32),
        pltpu.SemaphoreType.DMA,
        pltpu.SemaphoreType.DMA,
        pltpu.SemaphoreType.DMA,
    ],
)
def _p3c(*args):
    _p3c_body(args)


def _make_p3(NSL, CPH):
    @functools.partial(
        pl.kernel,
        out_type=tuple(jax.ShapeDtypeStruct((NC, NP, 16), jnp.float32)
                       for _ in range(NSL)),
        mesh=_MESH,
        compiler_params=_SC_PARAMS,
        scratch_types=[
            pltpu.VMEM((W,), jnp.int32),
            pltpu.VMEM((W,), jnp.int32),
            pltpu.VMEM((W,), jnp.int32),
            pltpu.VMEM((W,), jnp.int32),
            pltpu.VMEM((W2 * 4 + 16,), jnp.float32),
            pltpu.VMEM((W2, 16), jnp.float32),
            pltpu.VMEM((W2, 16), jnp.float32),
            pltpu.VMEM((832, 16), jnp.float32),
            pltpu.MemorySpace.VMEM_SHARED((NP, 16), jnp.float32),
            pltpu.SemaphoreType.DMA,
            pltpu.SemaphoreType.DMA,
            pltpu.SemaphoreType.DMA,
        ],
    )
    def _p3(*args):
        _p3_body(NSL, CPH, args)
    return _p3


def _pool_body(args):
    parts = args[:2]
    b_h, b2_h, g_h = args[2:5]
    b_v, b2_v, bufs, hrow_v, zb_v, spm, sem = args[5:]
    c, s, wid = _ids()
    pltpu.sync_copy(b2_h, b2_v)
    _zero_rows(zb_v, 32)
    pltpu.sync_copy(zb_v.at[pl.ds(0, GP // NS)], spm.at[pl.ds(s * (GP // NS), GP // NS)])
    plsc.subcore_barrier()

    def win(w, _):
        base = wid * RPW + w * W
        pltpu.sync_copy(b_h.at[pl.ds(base, W)], b_v)
        for t in range(2):
            for cc in range(NC):
                pltpu.sync_copy(parts[t].at[cc, pl.ds(base, W)], bufs[t * 2 + cc])

        def row(r, _):
            for jj in range(2):
                acc = bufs[2 * jj][r, pl.ds(0, 16)] + bufs[2 * jj + 1][r, pl.ds(0, 16)]
                hrow_v[r, pl.ds(jj * 16, 16)] = acc * 0.25 + b2_v[pl.ds(jj * 16, 16)]
            return 0

        lax.fori_loop(0, W, row, 0)
        pltpu.sync_copy(hrow_v, spm.at[b_v], add=True)
        return 0

    lax.fori_loop(0, RPW // W, win, 0)
    plsc.subcore_barrier()
    pltpu.sync_copy(spm.at[pl.ds(s * (GP // NS), GP // NS)],
                    g_h.at[c, pl.ds(s * (GP // NS), GP // NS)])


@functools.partial(
    pl.kernel,
    out_type=jax.ShapeDtypeStruct((NC, GP, 32), jnp.float32),
    mesh=_MESH,
    compiler_params=_SC_PARAMS,
    scratch_types=[
        pltpu.VMEM((W,), jnp.int32),
        pltpu.VMEM((32,), jnp.float32),
        [pltpu.VMEM((W, 16), jnp.float32)] * 4,
        pltpu.VMEM((W, 32), jnp.float32),
        pltpu.VMEM((832, 32), jnp.float32),
        pltpu.MemorySpace.VMEM_SHARED((GP, 32), jnp.float32),
        pltpu.SemaphoreType.DMA,
    ],
)
def _pool(*args):
    _pool_body(args)


_R = 512
_GRID = NP // _R


def _tc1_body(x_ref, wl_ref, wr_ref, s0_ref, s1_ref,
              xl_ref, t0_ref, t1_ref, t2_ref, t3_ref, xr_ref, la_ref):
    xb = x_ref[...]
    xl = jnp.dot(xb, wl_ref[...], preferred_element_type=jnp.float32)
    xl_ref[...] = xl
    for h, t_ref in enumerate((t0_ref, t1_ref, t2_ref, t3_ref)):
        t_ref[...] = xl[:, 16 * h:16 * h + 16]
    xr_ref[...] = jnp.dot(xb, wr_ref[...], preferred_element_type=jnp.float32)
    ssum = s0_ref[...] + s1_ref[...]
    la_ref[...] = ssum / jnp.clip(ssum[:, 4:5], 1.0, None)


def _tc1(x_p, Wl1, Wr1, sums):
    return pl.pallas_call(
        _tc1_body,
        grid=(_GRID,),
        in_specs=[
            pl.BlockSpec((_R, 32), lambda i: (i, 0)),
            pl.BlockSpec((32, 64), lambda i: (0, 0)),
            pl.BlockSpec((32, 64), lambda i: (0, 0)),
            pl.BlockSpec((_R, 16), lambda i: (i, 0)),
            pl.BlockSpec((_R, 16), lambda i: (i, 0)),
        ],
        out_specs=[pl.BlockSpec((_R, 64), lambda i: (i, 0))]
        + [pl.BlockSpec((_R, 16), lambda i: (i, 0))] * 4
        + [pl.BlockSpec((_R, 64), lambda i: (i, 0)),
           pl.BlockSpec((_R, 16), lambda i: (i, 0))],
        out_shape=[jax.ShapeDtypeStruct((NP, 64), jnp.float32)]
        + [jax.ShapeDtypeStruct((NP, 16), jnp.float32)] * 4
        + [jax.ShapeDtypeStruct((NP, 64), jnp.float32),
           jax.ShapeDtypeStruct((NP, 16), jnp.float32)],
    )(x_p, Wl1, Wr1, sums[0], sums[1])


def _tc2_body(p0_ref, p1_ref, p2_ref, p3_ref, d_ref, b1_ref, wl_ref, wr_ref,
              xl_ref, xr_ref):
    d = d_ref[0] + d_ref[1]
    segs = [(p_ref[0] + p_ref[1]) / (d[:, h:h + 1] + 1e-16)
            for h, p_ref in enumerate((p0_ref, p1_ref, p2_ref, p3_ref))]
    h = jnp.maximum(jnp.concatenate(segs, axis=1) + b1_ref[...], 0.0)
    xl_ref[...] = jnp.dot(h, wl_ref[...], preferred_element_type=jnp.float32)
    xr_ref[...] = jnp.dot(h, wr_ref[...], preferred_element_type=jnp.float32)


def _tc2(o1, den1, b1, Wl2, Wr2):
    return pl.pallas_call(
        _tc2_body,
        grid=(_GRID,),
        in_specs=[pl.BlockSpec((2, _R, 16), lambda i: (0, i, 0))] * 5
        + [
            pl.BlockSpec((1, 64), lambda i: (0, 0)),
            pl.BlockSpec((64, 128), lambda i: (0, 0)),
            pl.BlockSpec((64, 128), lambda i: (0, 0)),
        ],
        out_specs=[pl.BlockSpec((_R, 128), lambda i: (i, 0))] * 2,
        out_shape=[jax.ShapeDtypeStruct((NP, 128), jnp.float32)] * 2,
    )(o1[0], o1[1], o1[2], o1[3], den1, b1.reshape(1, 64), Wl2, Wr2)


_EB = EP // _GRID


def _ef_body(ea_ref, we_ref, o_ref):
    o_ref[...] = jnp.dot(ea_ref[...], we_ref[...],
                         preferred_element_type=jnp.float32)


def _ef(ea2d, We):
    HC = We.shape[1]
    return pl.pallas_call(
        _ef_body,
        grid=(_GRID,),
        in_specs=[pl.BlockSpec((_EB, 4), lambda i: (i, 0)),
                  pl.BlockSpec((4, HC), lambda i: (0, 0))],
        out_specs=pl.BlockSpec((_EB, HC), lambda i: (i, 0)),
        out_shape=jax.ShapeDtypeStruct((EP, HC), jnp.float32),
    )(ea2d, We)


def _merge_body(a_ref, b_ref, o_ref):
    o_ref[...] = a_ref[...] + b_ref[...]


def _merge16(den):
    return pl.pallas_call(
        _merge_body,
        grid=(_GRID,),
        in_specs=[pl.BlockSpec((_R, 16), lambda i: (i, 0)),
                  pl.BlockSpec((_R, 16), lambda i: (i, 0))],
        out_specs=pl.BlockSpec((_R, 16), lambda i: (i, 0)),
        out_shape=jax.ShapeDtypeStruct((NP, 16), jnp.float32),
    )(den[0], den[1])


def _tc4_body(g_ref, w3_ref, b3_ref, gm_ref, bt_ref, w4_ref, b4_ref, o_ref):
    g = g_ref[0, :G, :] + g_ref[1, :G, :]
    h = jnp.maximum(jnp.dot(g, w3_ref[...], preferred_element_type=jnp.float32)
                    + b3_ref[...], 0.0)
    mu = jnp.mean(h, axis=-1, keepdims=True)
    var = jnp.mean((h - mu) ** 2, axis=-1, keepdims=True)
    hn = (h - mu) * jax.lax.rsqrt(var + 1e-5) * gm_ref[...] + bt_ref[...]
    o_ref[...] = jnp.dot(hn, w4_ref[...], preferred_element_type=jnp.float32) + b4_ref[...]


def _tc4(gacc, W3, b3, gamma, beta, W4, b4):
    return pl.pallas_call(
        _tc4_body,
        out_shape=jax.ShapeDtypeStruct((G, 64), jnp.float32),
    )(gacc, W3, b3.reshape(1, 128), gamma.reshape(1, 128),
      beta.reshape(1, 128), W4, b4.reshape(1, 64))


_P1_L1 = _make_p1(64, 16, W2)
_P1_L2 = _make_p1(128, 32, W)
_P3_L1 = _make_p3(4, 1)


def kernel(x, edge_index, edge_attr, batch, Wl1, Wr1, We1, att1, b1,
           Wl2, Wr2, We2, att2, b2, W3, b3, gamma, beta, W4, b4):
    src0 = edge_index[0]
    dst0 = edge_index[1]

    pad0 = E0P - E
    dst0_p = jnp.concatenate([dst0, N + (jnp.arange(pad0, dtype=jnp.int32) % 16)])
    ea0_f = jnp.concatenate(
        [edge_attr, jnp.zeros((pad0, 4), jnp.float32)]).reshape(-1)

    x_p = jnp.pad(x, ((0, NP - N), (0, 0)))

    sums = _p0(dst0_p, ea0_f)
    xl1, t0, t1, t2, t3, xr1, la = _tc1(x_p, Wl1, Wr1, sums)

    loop_attr = la[:N, :4]
    pad1 = EP - E - N
    loop_idx = jnp.arange(N, dtype=jnp.int32)
    pad_idx = N + (jnp.arange(pad1, dtype=jnp.int32) % 16)
    src_p = jnp.concatenate([src0, loop_idx, pad_idx])
    dst_p = jnp.concatenate([dst0, loop_idx, pad_idx])
    ea_f = jnp.concatenate(
        [edge_attr, loop_attr, jnp.zeros((pad1, 4), jnp.float32)]).reshape(-1)

    ea2d = ea_f.reshape(EP, 4)
    ef1 = _ef(ea2d, We1)
    ef2 = _ef(ea2d, We2)
    ex1, den1 = _P1_L1(xl1, xr1, ef1, src_p, dst_p, att1)
    o1 = _P3_L1(t0, t1, t2, t3, src_p, dst_p, ex1)

    xl2, xr2 = _tc2(o1, den1, b1, Wl2, Wr2)

    ex2, den2 = _P1_L2(xl2, xr2, ef2, src_p, dst_p, att2)
    den2m = _merge16(den2)
    o2 = _p3c(xl2, den2m, src_p, dst_p, ex2)

    batch_p = jnp.concatenate(
        [batch, G + (jnp.arange(NP - N, dtype=jnp.int32) % 32)])
    gacc = _pool(*o2, batch_p, b2)
    return _tc4(gacc, W3, b3, gamma, beta, W4, b4)
